# Initial kernel scaffold; baseline (speedup 1.0000x reference)
import functools

import numpy as np
import jax
import jax.numpy as jnp
from jax import lax
from jax.experimental import pallas as pl
from jax.experimental.pallas import tpu as pltpu
from jax.experimental.pallas import tpu_sc as plsc

FDIM = 128
KRBF = 5
CUTOFF = 10.0
GRP = 32
BN = 200
BE = BN * GRP
SC_CH = 80
SC_NW = 32
LN2 = np.float32(np.log(2.0))


def _act(x):
    return jax.nn.softplus(x) - LN2


@functools.lru_cache(maxsize=None)
def _sc_gather_fn(V, D, B):
    assert B % (SC_NW * SC_CH) == 0 and D % 16 == 0
    nchunks = B // (SC_NW * SC_CH)
    per_w = nchunks * SC_CH
    mesh = plsc.VectorSubcoreMesh(core_axis_name="c", subcore_axis_name="s")

    @functools.partial(
        pl.kernel,
        mesh=mesh,
        out_type=jax.ShapeDtypeStruct((B, D), jnp.float32),
        scratch_types=[
            pltpu.VMEM((SC_CH,), jnp.int32),
            pltpu.VMEM((SC_CH, D), jnp.float32),
            pltpu.SemaphoreType.DMA,
        ],
    )
    def k(table_hbm, idx_hbm, out_hbm, idx_v, rows_v, sem):
        wid = lax.axis_index("s") * 2 + lax.axis_index("c")
        base0 = wid * per_w

        def body(i, carry):
            base = base0 + i * SC_CH
            pltpu.sync_copy(idx_hbm.at[pl.ds(base, SC_CH)], idx_v)
            pltpu.async_copy(table_hbm.at[idx_v], rows_v, sem).wait()
            pltpu.sync_copy(rows_v, out_hbm.at[pl.ds(base, SC_CH)])
            return carry

        lax.fori_loop(0, nchunks, body, 0)

    return k


def _sc_gather(table, idx):
    V, D = table.shape
    B = idx.shape[0]
    return _sc_gather_fn(V, D, B)(table, idx)


_CENTERS = np.zeros((8,), np.float32)
_CENTERS[:KRBF] = np.linspace(1.0, np.exp(-CUTOFF), KRBF).astype(np.float32)
_KMASK = np.zeros((8,), np.float32)
_KMASK[:KRBF] = 1.0
_WIDTH = np.float32((0.5 / ((1.0 - np.exp(-CUTOFF)) / KRBF)) ** 2)


def _rbf_body(ri_ref, rj_ref, out_ref):
    d = ri_ref[...] - rj_ref[...]
    d2 = jnp.sum(d * d, axis=1, keepdims=True)
    D = jnp.sqrt(jnp.maximum(d2, 0.0))
    x = D / CUTOFF
    x3 = x ** 3
    x4 = x3 * x
    x5 = x4 * x
    cf = jnp.where(x < 1.0, 1.0 - 6.0 * x5 + 15.0 * x4 - 10.0 * x3,
                   jnp.zeros_like(x))
    cen = jnp.asarray(_CENTERS)
    msk = jnp.asarray(_KMASK)
    out_ref[...] = (cf * jnp.exp(-_WIDTH * (jnp.exp(-D) - cen) ** 2)) * msk


def _rbf_call(rif, rjf):
    E = rif.shape[0]
    grid = E // BE
    return pl.pallas_call(
        _rbf_body,
        grid=(grid,),
        in_specs=[
            pl.BlockSpec((BE, 16), lambda i: (i, 0)),
            pl.BlockSpec((BE, 16), lambda i: (i, 0)),
        ],
        out_specs=pl.BlockSpec((BE, 8), lambda i: (i, 0)),
        out_shape=jax.ShapeDtypeStruct((E, 8), jnp.float32),
    )(rif, rjf)


def _proj_body(x_ref, wi_ref, bi_ref, wj_ref, bj_ref, xi_ref, tj_ref):
    xa = _act(x_ref[...])
    xi_ref[...] = (jnp.dot(xa, wi_ref[...], preferred_element_type=jnp.float32)
                   + bi_ref[...])
    tj_ref[...] = (jnp.dot(xa, wj_ref[...], preferred_element_type=jnp.float32)
                   + bj_ref[...])


def _proj_call(x, wi, bi, wj, bj):
    N = x.shape[0]
    grid = N // BN
    full = lambda shape: pl.BlockSpec(shape, lambda i: tuple(0 for _ in shape))
    row = pl.BlockSpec((BN, FDIM), lambda i: (i, 0))
    return pl.pallas_call(
        _proj_body,
        grid=(grid,),
        in_specs=[row, full((FDIM, FDIM)), full((1, FDIM)),
                  full((FDIM, FDIM)), full((1, FDIM))],
        out_specs=[row, row],
        out_shape=[jax.ShapeDtypeStruct((N, FDIM), jnp.float32)] * 2,
    )(x, wi, bi, wj, bj)


_PERM = np.zeros((FDIM, FDIM), np.float32)
for _r in range(4):
    for _q in range(GRP):
        _PERM[4 * _q + _r, 32 * _r + _q] = 1.0


def _block_body(refs, *, fuse_next, do_nh):
    (x_ref, xi_ref, xg_ref, rbf_ref, k2f_ref,
     i1w, i1b, i2w, i2b, dw, db, u_ref,
     a1w, a1b, a2w, a2b, o1w, o1b, o2w, o2b, od_ref, p_ref) = refs[:22]
    pos = 22
    if fuse_next:
        wi2, bi2, wj2, bj2 = refs[pos:pos + 4]
        pos += 4
    xout_ref = refs[pos]
    out_ref = refs[pos + 1]
    pos += 2
    if fuse_next:
        xi2_ref, tj2_ref = refs[pos:pos + 2]
        pos += 2
    if do_nh:
        nh_ref = refs[pos]

    dot = lambda a, b: jnp.dot(a, b, preferred_element_type=jnp.float32)

    g = dot(rbf_ref[...], k2f_ref[...])
    xj = g * xg_ref[...]
    X = xj.reshape(BN, GRP, FDIM)
    xi = xi_ref[...]
    xiP = dot(xi, p_ref[...])

    att = jnp.zeros((BN, GRP), jnp.float32)
    for r in range(4):
        xs = X[:, :, 32 * r:32 * r + 32]
        xir = xiP[:, 32 * r:32 * r + 32].reshape(BN, GRP, 1)
        att = att + jnp.sum(xs * xir, axis=1)
    att = att - jnp.max(att, axis=1, keepdims=True)
    ea = jnp.exp(att)
    attw = ea / jnp.sum(ea, axis=1, keepdims=True)
    xjagg = jnp.sum(X * attw.reshape(BN, GRP, 1), axis=1)

    m = xi + xjagg
    m = m + dot(dot(_act(m), i1w[...]) + i1b[...], i2w[...]) + i2b[...]
    m = _act(m)
    xnew = u_ref[...] * x_ref[...] + dot(m, dw[...]) + db[...]
    xo = xnew + dot(dot(_act(xnew), a1w[...]) + a1b[...], a2w[...]) + a2b[...]
    h = xo + dot(dot(_act(xo), o1w[...]) + o1b[...], o2w[...]) + o2b[...]
    h = _act(h)
    out = dot(h, od_ref[...])

    xout_ref[...] = xo
    out_ref[...] = out
    if fuse_next:
        xa2 = _act(xo)
        xi2_ref[...] = dot(xa2, wi2[...]) + bi2[...]
        tj2_ref[...] = dot(xa2, wj2[...]) + bj2[...]
    if do_nh:
        o2 = out * out
        part = jnp.sum(o2 / (o2 + 1e-7)).reshape(1, 1)

        @pl.when(pl.program_id(0) == 0)
        def _():
            nh_ref[...] = jnp.zeros_like(nh_ref)

        nh_ref[...] += part


def _block_call(x, xi, xg, rbf, wts, nxt):
    N = x.shape[0]
    grid = N // BN
    fuse_next = nxt is not None
    do_nh = not fuse_next

    full = lambda shape: pl.BlockSpec(shape, lambda i: tuple(0 for _ in shape))
    row = pl.BlockSpec((BN, FDIM), lambda i: (i, 0))
    erow = pl.BlockSpec((BE, FDIM), lambda i: (i, 0))
    rrow = pl.BlockSpec((BE, 8), lambda i: (i, 0))
    orow = pl.BlockSpec((BN, 8), lambda i: (i, 0))

    in_specs = [row, row, erow, rrow, full((8, FDIM))]
    in_specs += [full((FDIM, FDIM)), full((1, FDIM))] * 2
    in_specs += [full((FDIM, FDIM)), full((1, FDIM)), full((1, FDIM))]
    in_specs += [full((FDIM, FDIM)), full((1, FDIM))] * 4
    in_specs += [full((FDIM, 8)), full((FDIM, FDIM))]
    if fuse_next:
        in_specs += [full((FDIM, FDIM)), full((1, FDIM))] * 2

    out_specs = [row, orow]
    out_shape = [jax.ShapeDtypeStruct((N, FDIM), jnp.float32),
                 jax.ShapeDtypeStruct((N, 8), jnp.float32)]
    if fuse_next:
        out_specs += [row, row]
        out_shape += [jax.ShapeDtypeStruct((N, FDIM), jnp.float32)] * 2
    if do_nh:
        out_specs += [pl.BlockSpec((1, 1), lambda i: (0, 0))]
        out_shape += [jax.ShapeDtypeStruct((1, 1), jnp.float32)]

    body = lambda *refs: _block_body(refs, fuse_next=fuse_next, do_nh=do_nh)
    args = [x, xi, xg, rbf] + wts + (nxt if fuse_next else [])
    return pl.pallas_call(
        body,
        grid=(grid,),
        in_specs=in_specs,
        out_specs=out_specs,
        out_shape=out_shape,
    )(*args)


def _block_weights(p):
    b = lambda v: v.reshape(1, FDIM)
    od = jnp.pad(p['odense'], ((0, 0), (0, 6)))
    k2f = jnp.pad(p['k2f'], ((0, 3), (0, 0)))
    rp = p['ires'][0]
    ap = p['ares'][0]
    op = p['ores'][0]
    return [k2f,
            rp['d1']['w'], b(rp['d1']['b']), rp['d2']['w'], b(rp['d2']['b']),
            p['dense']['w'], b(p['dense']['b']), b(p['u']),
            ap['d1']['w'], b(ap['d1']['b']), ap['d2']['w'], b(ap['d2']['b']),
            op['d1']['w'], b(op['d1']['b']), op['d2']['w'], b(op['d2']['b']),
            od, jnp.asarray(_PERM)]


def kernel(Z, R, idx_i, idx_j, params):
    N = Z.shape[0]
    Z = Z.astype(jnp.int32)
    idx_i = idx_i.astype(jnp.int32)
    idx_j = idx_j.astype(jnp.int32)

    Rp = jnp.pad(R.astype(jnp.float32), ((0, 0), (0, 13)))
    rif = _sc_gather(Rp, idx_i)
    rjf = _sc_gather(Rp, idx_j)
    rbf = _rbf_call(rif, rjf)

    npad = (-N) % (SC_NW * SC_CH)
    x = _sc_gather(params['emb'], jnp.pad(Z, (0, npad)))[:N]

    b1, b2 = params['blocks']
    xi1, tj1 = _proj_call(x, b1['di']['w'], b1['di']['b'].reshape(1, FDIM),
                          b1['dj']['w'], b1['dj']['b'].reshape(1, FDIM))
    xg1 = _sc_gather(tj1, idx_j)
    nxt = [b2['di']['w'], b2['di']['b'].reshape(1, FDIM),
           b2['dj']['w'], b2['dj']['b'].reshape(1, FDIM)]
    x1, out1, xi2, tj2 = _block_call(x, xi1, xg1, rbf, _block_weights(b1), nxt)
    xg2 = _sc_gather(tj2, idx_j)
    x2, out2, nh = _block_call(x1, xi2, xg2, rbf, _block_weights(b2), None)

    e_total = out1[:, 0] + out2[:, 0]
    q_total = out1[:, 1] + out2[:, 1]
    nhloss = nh[0, 0] / np.float32(N * 2)
    return (e_total, q_total, nhloss)


# speedup vs baseline: 2.8217x; 2.8217x over previous
import functools

import numpy as np
import jax
import jax.numpy as jnp
from jax import lax
from jax.experimental import pallas as pl
from jax.experimental.pallas import tpu as pltpu
from jax.experimental.pallas import tpu_sc as plsc

FDIM = 128
KRBF = 5
CUTOFF = 10.0
GRP = 32
BN = 200
BE = BN * GRP
SC_CH = 80
SC_NW = 32
LN2 = np.float32(np.log(2.0))


def _act(x):
    return jax.nn.softplus(x) - LN2


@functools.lru_cache(maxsize=None)
def _sc_gather_fn(V, D, B):
    assert B % (SC_NW * SC_CH) == 0 and D % 16 == 0
    nchunks = B // (SC_NW * SC_CH)
    per_w = nchunks * SC_CH
    mesh = plsc.VectorSubcoreMesh(core_axis_name="c", subcore_axis_name="s")

    @functools.partial(
        pl.kernel,
        mesh=mesh,
        out_type=jax.ShapeDtypeStruct((B, D), jnp.float32),
        scratch_types=[
            pltpu.VMEM((SC_CH,), jnp.int32),
            pltpu.VMEM((SC_CH, D), jnp.float32),
            pltpu.SemaphoreType.DMA,
        ],
    )
    def k(table_hbm, idx_hbm, out_hbm, idx_v, rows_v, sem):
        wid = lax.axis_index("s") * 2 + lax.axis_index("c")
        base0 = wid * per_w

        def body(i, carry):
            base = base0 + i * SC_CH
            pltpu.sync_copy(idx_hbm.at[pl.ds(base, SC_CH)], idx_v)
            pltpu.async_copy(table_hbm.at[idx_v], rows_v, sem).wait()
            pltpu.sync_copy(rows_v, out_hbm.at[pl.ds(base, SC_CH)])
            return carry

        lax.fori_loop(0, nchunks, body, 0)

    return k


def _sc_gather(table, idx):
    V, D = table.shape
    B = idx.shape[0]
    return _sc_gather_fn(V, D, B)(table, idx)


@functools.lru_cache(maxsize=None)
def _sc_d2_fn(N, E):
    assert E % (SC_NW * 16) == 0
    per_w = E // SC_NW
    nv = per_w // 16
    mesh = plsc.VectorSubcoreMesh(core_axis_name="c", subcore_axis_name="s")

    @functools.partial(
        pl.kernel,
        mesh=mesh,
        compiler_params=pltpu.CompilerParams(needs_layout_passes=False),
        out_type=jax.ShapeDtypeStruct((E,), jnp.float32),
        scratch_types=[
            pltpu.VMEM((N,), jnp.float32),
            pltpu.VMEM((N,), jnp.float32),
            pltpu.VMEM((N,), jnp.float32),
            pltpu.VMEM((per_w,), jnp.int32),
            pltpu.VMEM((per_w,), jnp.int32),
            pltpu.VMEM((per_w,), jnp.float32),
        ],
    )
    def k(rx_h, ry_h, rz_h, ii_h, ij_h, out_h, rx, ry, rz, iiv, ijv, d2v):
        wid = lax.axis_index("s") * 2 + lax.axis_index("c")
        base = wid * per_w
        pltpu.sync_copy(rx_h, rx)
        pltpu.sync_copy(ry_h, ry)
        pltpu.sync_copy(rz_h, rz)
        pltpu.sync_copy(ii_h.at[pl.ds(base, per_w)], iiv)
        pltpu.sync_copy(ij_h.at[pl.ds(base, per_w)], ijv)

        def body(v, carry):
            s = pl.ds(v * 16, 16)
            ii = iiv[s]
            ij = ijv[s]
            dx = plsc.load_gather(rx, [ii]) - plsc.load_gather(rx, [ij])
            dy = plsc.load_gather(ry, [ii]) - plsc.load_gather(ry, [ij])
            dz = plsc.load_gather(rz, [ii]) - plsc.load_gather(rz, [ij])
            d2v[s] = dx * dx + dy * dy + dz * dz
            return carry

        lax.fori_loop(0, nv, body, 0)
        pltpu.sync_copy(d2v, out_h.at[pl.ds(base, per_w)])

    return k


def _sc_edge_d2(R, idx_i, idx_j):
    N = R.shape[0]
    E = idx_i.shape[0]
    return _sc_d2_fn(N, E)(R[:, 0], R[:, 1], R[:, 2], idx_i, idx_j)


_CSTEP = float((np.exp(-CUTOFF) - 1.0) / (KRBF - 1))
_WIDTH = np.float32((0.5 / ((1.0 - np.exp(-CUTOFF)) / KRBF)) ** 2)


def _rbf_body(d2_ref, out_ref):
    D = jnp.sqrt(jnp.maximum(d2_ref[...], 0.0))
    x = D / CUTOFF
    x3 = x ** 3
    x4 = x3 * x
    x5 = x4 * x
    cf = jnp.where(x < 1.0, 1.0 - 6.0 * x5 + 15.0 * x4 - 10.0 * x3,
                   jnp.zeros_like(x))
    k = lax.broadcasted_iota(jnp.int32, (1, 8), 1).astype(jnp.float32)
    cen = jnp.where(k < KRBF, 1.0 + k * _CSTEP, 0.0)
    msk = (k < KRBF).astype(jnp.float32)
    out_ref[...] = (cf * jnp.exp(-_WIDTH * (jnp.exp(-D) - cen) ** 2)) * msk


def _rbf_call(d2):
    E = d2.shape[0]
    grid = E // BE
    return pl.pallas_call(
        _rbf_body,
        grid=(grid,),
        in_specs=[pl.BlockSpec((BE, 1), lambda i: (i, 0))],
        out_specs=pl.BlockSpec((BE, 8), lambda i: (i, 0)),
        out_shape=jax.ShapeDtypeStruct((E, 8), jnp.float32),
    )(d2.reshape(E, 1))


def _proj_body(x_ref, wi_ref, bi_ref, wj_ref, bj_ref, xi_ref, tj_ref):
    xa = _act(x_ref[...])
    xi_ref[...] = (jnp.dot(xa, wi_ref[...], preferred_element_type=jnp.float32)
                   + bi_ref[...])
    tj_ref[...] = (jnp.dot(xa, wj_ref[...], preferred_element_type=jnp.float32)
                   + bj_ref[...])


def _proj_call(x, wi, bi, wj, bj):
    N = x.shape[0]
    grid = N // BN
    full = lambda shape: pl.BlockSpec(shape, lambda i: tuple(0 for _ in shape))
    row = pl.BlockSpec((BN, FDIM), lambda i: (i, 0))
    return pl.pallas_call(
        _proj_body,
        grid=(grid,),
        in_specs=[row, full((FDIM, FDIM)), full((1, FDIM)),
                  full((FDIM, FDIM)), full((1, FDIM))],
        out_specs=[row, row],
        out_shape=[jax.ShapeDtypeStruct((N, FDIM), jnp.float32)] * 2,
    )(x, wi, bi, wj, bj)


_PERM = np.zeros((FDIM, FDIM), np.float32)
for _r in range(4):
    for _q in range(GRP):
        _PERM[4 * _q + _r, 32 * _r + _q] = 1.0


def _block_body(refs, *, fuse_next, do_nh):
    (x_ref, xi_ref, xg_ref, rbf_ref, k2f_ref,
     i1w, i1b, i2w, i2b, dw, db, u_ref,
     a1w, a1b, a2w, a2b, o1w, o1b, o2w, o2b, od_ref, p_ref) = refs[:22]
    pos = 22
    if fuse_next:
        wi2, bi2, wj2, bj2 = refs[pos:pos + 4]
        pos += 4
    xout_ref = refs[pos]
    out_ref = refs[pos + 1]
    pos += 2
    if fuse_next:
        xi2_ref, tj2_ref = refs[pos:pos + 2]
        pos += 2
    if do_nh:
        nh_ref = refs[pos]

    dot = lambda a, b: jnp.dot(a, b, preferred_element_type=jnp.float32)

    g = dot(rbf_ref[...], k2f_ref[...])
    xj = g * xg_ref[...]
    X = xj.reshape(BN, GRP, FDIM)
    xi = xi_ref[...]
    xiP = dot(xi, p_ref[...])

    att = jnp.zeros((BN, GRP), jnp.float32)
    for r in range(4):
        xs = X[:, :, 32 * r:32 * r + 32]
        xir = xiP[:, 32 * r:32 * r + 32].reshape(BN, GRP, 1)
        att = att + jnp.sum(xs * xir, axis=1)
    att = att - jnp.max(att, axis=1, keepdims=True)
    ea = jnp.exp(att)
    attw = ea / jnp.sum(ea, axis=1, keepdims=True)
    xjagg = jnp.sum(X * attw.reshape(BN, GRP, 1), axis=1)

    m = xi + xjagg
    m = m + dot(dot(_act(m), i1w[...]) + i1b[...], i2w[...]) + i2b[...]
    m = _act(m)
    xnew = u_ref[...] * x_ref[...] + dot(m, dw[...]) + db[...]
    xo = xnew + dot(dot(_act(xnew), a1w[...]) + a1b[...], a2w[...]) + a2b[...]
    h = xo + dot(dot(_act(xo), o1w[...]) + o1b[...], o2w[...]) + o2b[...]
    h = _act(h)
    out = dot(h, od_ref[...])

    xout_ref[...] = xo
    out_ref[...] = out
    if fuse_next:
        xa2 = _act(xo)
        xi2_ref[...] = dot(xa2, wi2[...]) + bi2[...]
        tj2_ref[...] = dot(xa2, wj2[...]) + bj2[...]
    if do_nh:
        o2 = out * out
        part = jnp.sum(o2 / (o2 + 1e-7)).reshape(1, 1)

        @pl.when(pl.program_id(0) == 0)
        def _():
            nh_ref[...] = jnp.zeros_like(nh_ref)

        nh_ref[...] += part


def _block_call(x, xi, xg, rbf, wts, nxt):
    N = x.shape[0]
    grid = N // BN
    fuse_next = nxt is not None
    do_nh = not fuse_next

    full = lambda shape: pl.BlockSpec(shape, lambda i: tuple(0 for _ in shape))
    row = pl.BlockSpec((BN, FDIM), lambda i: (i, 0))
    erow = pl.BlockSpec((BE, FDIM), lambda i: (i, 0))
    rrow = pl.BlockSpec((BE, 8), lambda i: (i, 0))
    orow = pl.BlockSpec((BN, 8), lambda i: (i, 0))

    in_specs = [row, row, erow, rrow, full((8, FDIM))]
    in_specs += [full((FDIM, FDIM)), full((1, FDIM))] * 2
    in_specs += [full((FDIM, FDIM)), full((1, FDIM)), full((1, FDIM))]
    in_specs += [full((FDIM, FDIM)), full((1, FDIM))] * 4
    in_specs += [full((FDIM, 8)), full((FDIM, FDIM))]
    if fuse_next:
        in_specs += [full((FDIM, FDIM)), full((1, FDIM))] * 2

    out_specs = [row, orow]
    out_shape = [jax.ShapeDtypeStruct((N, FDIM), jnp.float32),
                 jax.ShapeDtypeStruct((N, 8), jnp.float32)]
    if fuse_next:
        out_specs += [row, row]
        out_shape += [jax.ShapeDtypeStruct((N, FDIM), jnp.float32)] * 2
    if do_nh:
        out_specs += [pl.BlockSpec((1, 1), lambda i: (0, 0))]
        out_shape += [jax.ShapeDtypeStruct((1, 1), jnp.float32)]

    body = lambda *refs: _block_body(refs, fuse_next=fuse_next, do_nh=do_nh)
    args = [x, xi, xg, rbf] + wts + (nxt if fuse_next else [])
    return pl.pallas_call(
        body,
        grid=(grid,),
        in_specs=in_specs,
        out_specs=out_specs,
        out_shape=out_shape,
    )(*args)


def _block_weights(p):
    b = lambda v: v.reshape(1, FDIM)
    od = jnp.pad(p['odense'], ((0, 0), (0, 6)))
    k2f = jnp.pad(p['k2f'], ((0, 3), (0, 0)))
    rp = p['ires'][0]
    ap = p['ares'][0]
    op = p['ores'][0]
    return [k2f,
            rp['d1']['w'], b(rp['d1']['b']), rp['d2']['w'], b(rp['d2']['b']),
            p['dense']['w'], b(p['dense']['b']), b(p['u']),
            ap['d1']['w'], b(ap['d1']['b']), ap['d2']['w'], b(ap['d2']['b']),
            op['d1']['w'], b(op['d1']['b']), op['d2']['w'], b(op['d2']['b']),
            od, jnp.asarray(_PERM)]


def kernel(Z, R, idx_i, idx_j, params):
    N = Z.shape[0]
    Z = Z.astype(jnp.int32)
    idx_i = idx_i.astype(jnp.int32)
    idx_j = idx_j.astype(jnp.int32)

    d2 = _sc_edge_d2(R.astype(jnp.float32), idx_i, idx_j)
    rbf = _rbf_call(d2)

    npad = (-N) % (SC_NW * SC_CH)
    x = _sc_gather(params['emb'], jnp.pad(Z, (0, npad)))[:N]

    b1, b2 = params['blocks']
    xi1, tj1 = _proj_call(x, b1['di']['w'], b1['di']['b'].reshape(1, FDIM),
                          b1['dj']['w'], b1['dj']['b'].reshape(1, FDIM))
    xg1 = _sc_gather(tj1, idx_j)
    nxt = [b2['di']['w'], b2['di']['b'].reshape(1, FDIM),
           b2['dj']['w'], b2['dj']['b'].reshape(1, FDIM)]
    x1, out1, xi2, tj2 = _block_call(x, xi1, xg1, rbf, _block_weights(b1), nxt)
    xg2 = _sc_gather(tj2, idx_j)
    x2, out2, nh = _block_call(x1, xi2, xg2, rbf, _block_weights(b2), None)

    e_total = out1[:, 0] + out2[:, 0]
    q_total = out1[:, 1] + out2[:, 1]
    nhloss = nh[0, 0] / np.float32(N * 2)
    return (e_total, q_total, nhloss)


# speedup vs baseline: 4.3206x; 1.5312x over previous
import functools

import numpy as np
import jax
import jax.numpy as jnp
from jax import lax
from jax.experimental import pallas as pl
from jax.experimental.pallas import tpu as pltpu
from jax.experimental.pallas import tpu_sc as plsc

FDIM = 128
KRBF = 5
CUTOFF = 10.0
GRP = 32
NSLICE = 5
BN = 400
BE = BN * GRP
SC_CH = 80
SC_NW = 32
LN2 = np.float32(np.log(2.0))
_CSTEP = float((np.exp(-CUTOFF) - 1.0) / (KRBF - 1))
_WIDTH = np.float32((0.5 / ((1.0 - np.exp(-CUTOFF)) / KRBF)) ** 2)


def _act(x):
    return jax.nn.softplus(x) - LN2


@functools.lru_cache(maxsize=None)
def _sc_gather_fn(V, D, B):
    assert B % SC_CH == 0 and D % 128 == 0
    nch = B // SC_CH
    iters = (nch + SC_NW - 1) // SC_NW
    mesh = plsc.VectorSubcoreMesh(core_axis_name="c", subcore_axis_name="s")

    @functools.partial(
        pl.kernel,
        mesh=mesh,
        out_type=jax.ShapeDtypeStruct((B, D), jnp.float32),
        scratch_types=[
            pltpu.VMEM((SC_CH,), jnp.int32),
            pltpu.VMEM((SC_CH, D), jnp.float32),
            pltpu.SemaphoreType.DMA,
        ],
    )
    def k(table_hbm, idx_hbm, out_hbm, idx_v, rows_v, sem):
        wid = lax.axis_index("s") * 2 + lax.axis_index("c")

        def body(i, carry):
            c = i * SC_NW + wid

            @pl.when(c < nch)
            def _():
                base = c * SC_CH
                pltpu.sync_copy(idx_hbm.at[pl.ds(base, SC_CH)], idx_v)
                pltpu.async_copy(table_hbm.at[idx_v], rows_v, sem).wait()
                pltpu.sync_copy(rows_v, out_hbm.at[pl.ds(base, SC_CH)])

            return carry

        lax.fori_loop(0, iters, body, 0)

    return k


def _sc_gather(table, idx):
    V, D = table.shape
    B = idx.shape[0]
    return _sc_gather_fn(V, D, B)(table, idx)


@functools.lru_cache(maxsize=None)
def _sc_d2_fn(N, E):
    assert E % (SC_NW * 16) == 0
    per_w = E // SC_NW
    nv = per_w // 16
    mesh = plsc.VectorSubcoreMesh(core_axis_name="c", subcore_axis_name="s")

    @functools.partial(
        pl.kernel,
        mesh=mesh,
        compiler_params=pltpu.CompilerParams(needs_layout_passes=False),
        out_type=jax.ShapeDtypeStruct((E,), jnp.float32),
        scratch_types=[
            pltpu.VMEM((N,), jnp.float32),
            pltpu.VMEM((N,), jnp.float32),
            pltpu.VMEM((N,), jnp.float32),
            pltpu.VMEM((per_w,), jnp.int32),
            pltpu.VMEM((per_w,), jnp.int32),
            pltpu.VMEM((per_w,), jnp.float32),
        ],
    )
    def k(rx_h, ry_h, rz_h, ii_h, ij_h, out_h, rx, ry, rz, iiv, ijv, d2v):
        wid = lax.axis_index("s") * 2 + lax.axis_index("c")
        base = wid * per_w
        pltpu.sync_copy(rx_h, rx)
        pltpu.sync_copy(ry_h, ry)
        pltpu.sync_copy(rz_h, rz)
        pltpu.sync_copy(ii_h.at[pl.ds(base, per_w)], iiv)
        pltpu.sync_copy(ij_h.at[pl.ds(base, per_w)], ijv)

        def body(v, carry):
            s = pl.ds(v * 16, 16)
            ii = iiv[s]
            ij = ijv[s]
            dx = plsc.load_gather(rx, [ii]) - plsc.load_gather(rx, [ij])
            dy = plsc.load_gather(ry, [ii]) - plsc.load_gather(ry, [ij])
            dz = plsc.load_gather(rz, [ii]) - plsc.load_gather(rz, [ij])
            d2v[s] = dx * dx + dy * dy + dz * dz
            return carry

        lax.fori_loop(0, nv, body, 0)
        pltpu.sync_copy(d2v, out_h.at[pl.ds(base, per_w)])

    return k


def _sc_edge_d2(R, idx_i, idx_j):
    N = R.shape[0]
    E = idx_i.shape[0]
    return _sc_d2_fn(N, E)(R[:, 0], R[:, 1], R[:, 2], idx_i, idx_j)


def _rbf_body(d2_ref, out_ref):
    d2 = d2_ref[...]
    D = jnp.sqrt(jnp.maximum(d2, 0.0))
    x = D / CUTOFF
    x3 = x ** 3
    x4 = x3 * x
    x5 = x4 * x
    cf = jnp.where(x < 1.0, 1.0 - 6.0 * x5 + 15.0 * x4 - 10.0 * x3,
                   jnp.zeros_like(x))
    eD = jnp.exp(-D)
    kk = lax.broadcasted_iota(jnp.int32, (8, 1, 1), 0)
    cen = jnp.where(kk < KRBF, 1.0 + kk.astype(jnp.float32) * _CSTEP, 0.0)
    msk = (kk < KRBF).astype(jnp.float32)
    val = cf[None] * jnp.exp(-_WIDTH * (eD[None] - cen) ** 2) * msk
    out_ref[...] = val.reshape(8, d2.size)


def _rbf_call(d2):
    E = d2.shape[0]
    return pl.pallas_call(
        _rbf_body,
        out_shape=jax.ShapeDtypeStruct((8, E), jnp.float32),
    )(d2.reshape(E // 128, 128))


def _proj_body(z_ref, emb_ref, wi_ref, bi_ref, wj_ref, bj_ref,
               x_ref, xi_ref, tj_ref):
    z = z_ref[...]
    oh = (z == lax.broadcasted_iota(jnp.int32, (z.shape[0], 32), 1))
    x = jnp.dot(oh.astype(jnp.float32), emb_ref[...],
                preferred_element_type=jnp.float32)
    xa = _act(x)
    x_ref[...] = x
    xi_ref[...] = (jnp.dot(xa, wi_ref[...], preferred_element_type=jnp.float32)
                   + bi_ref[...])
    tj_ref[...] = (jnp.dot(xa, wj_ref[...], preferred_element_type=jnp.float32)
                   + bj_ref[...])


def _proj_call(Z, emb, wi, bi, wj, bj):
    N = Z.shape[0]
    grid = N // BN
    full = lambda shape: pl.BlockSpec(shape, lambda i: tuple(0 for _ in shape))
    row = pl.BlockSpec((BN, FDIM), lambda i: (i, 0))
    embp = jnp.pad(emb, ((0, 32 - emb.shape[0]), (0, 0)))
    return pl.pallas_call(
        _proj_body,
        grid=(grid,),
        in_specs=[pl.BlockSpec((BN, 1), lambda i: (i, 0)), full((32, FDIM)),
                  full((FDIM, FDIM)), full((1, FDIM)),
                  full((FDIM, FDIM)), full((1, FDIM))],
        out_specs=[row, row, row],
        out_shape=[jax.ShapeDtypeStruct((N, FDIM), jnp.float32)] * 3,
    )(Z.reshape(N, 1), embp, wi, bi, wj, bj)


_PERM = np.zeros((FDIM, FDIM), np.float32)
for _r in range(4):
    for _q in range(GRP):
        _PERM[4 * _q + _r, 32 * _r + _q] = 1.0


def _block_body(refs, *, fuse_next, do_nh):
    (x_ref, xi_ref, xg_ref, rbf_ref, k2f_ref,
     i1w, i1b, i2w, i2b, dw, db, u_ref,
     a1w, a1b, a2w, a2b, o1w, o1b, o2w, o2b, od_ref, p_ref) = refs[:22]
    pos = 22
    if fuse_next:
        wi2, bi2, wj2, bj2 = refs[pos:pos + 4]
        pos += 4
    xout_ref = refs[pos]
    out_ref = refs[pos + 1]
    pos += 2
    if fuse_next:
        xi2_ref, tj2_ref = refs[pos:pos + 2]
        pos += 2
    if do_nh:
        nh_ref = refs[pos]

    dot = lambda a, b: jnp.dot(a, b, preferred_element_type=jnp.float32)

    g = lax.dot_general(rbf_ref[...], k2f_ref[...],
                        (((0,), (0,)), ((), ())),
                        preferred_element_type=jnp.float32)
    xj = g * xg_ref[...]
    X = xj.reshape(BN, GRP, FDIM)
    xi = xi_ref[...]
    xiP = dot(xi, p_ref[...])

    att = jnp.zeros((BN, GRP), jnp.float32)
    for r in range(4):
        xs = X[:, :, 32 * r:32 * r + 32]
        xir = xiP[:, 32 * r:32 * r + 32].reshape(BN, GRP, 1)
        att = att + jnp.sum(xs * xir, axis=1)
    att = att - jnp.max(att, axis=1, keepdims=True)
    ea = jnp.exp(att)
    attw = ea / jnp.sum(ea, axis=1, keepdims=True)
    xjagg = jnp.sum(X * attw.reshape(BN, GRP, 1), axis=1)

    m = xi + xjagg
    m = m + dot(dot(_act(m), i1w[...]) + i1b[...], i2w[...]) + i2b[...]
    m = _act(m)
    xnew = u_ref[...] * x_ref[...] + dot(m, dw[...]) + db[...]
    xo = xnew + dot(dot(_act(xnew), a1w[...]) + a1b[...], a2w[...]) + a2b[...]
    h = xo + dot(dot(_act(xo), o1w[...]) + o1b[...], o2w[...]) + o2b[...]
    h = _act(h)
    out = dot(h, od_ref[...])

    xout_ref[...] = xo
    out_ref[...] = out
    if fuse_next:
        xa2 = _act(xo)
        xi2_ref[...] = dot(xa2, wi2[...]) + bi2[...]
        tj2_ref[...] = dot(xa2, wj2[...]) + bj2[...]
    if do_nh:
        o2 = out * out
        part = jnp.sum(o2 / (o2 + 1e-7)).reshape(1, 1)

        @pl.when(pl.program_id(0) == 0)
        def _():
            nh_ref[...] = jnp.zeros_like(nh_ref)

        nh_ref[...] += part


def _block_call(x, xi, xg, rbf, wts, nxt, node0, ns):
    grid = ns // BN
    ro = node0 // BN
    fuse_next = nxt is not None
    do_nh = not fuse_next

    full = lambda shape: pl.BlockSpec(shape, lambda i: tuple(0 for _ in shape))
    rowo = pl.BlockSpec((BN, FDIM), lambda i: (i + ro, 0))
    row = pl.BlockSpec((BN, FDIM), lambda i: (i, 0))
    erow = pl.BlockSpec((BE, FDIM), lambda i: (i, 0))
    rrow = pl.BlockSpec((8, BE), lambda i: (0, i + ro))
    orow = pl.BlockSpec((BN, 8), lambda i: (i, 0))

    in_specs = [rowo, rowo, erow, rrow, full((8, FDIM))]
    in_specs += [full((FDIM, FDIM)), full((1, FDIM))] * 2
    in_specs += [full((FDIM, FDIM)), full((1, FDIM)), full((1, FDIM))]
    in_specs += [full((FDIM, FDIM)), full((1, FDIM))] * 4
    in_specs += [full((FDIM, 8)), full((FDIM, FDIM))]
    if fuse_next:
        in_specs += [full((FDIM, FDIM)), full((1, FDIM))] * 2

    out_specs = [row, orow]
    out_shape = [jax.ShapeDtypeStruct((ns, FDIM), jnp.float32),
                 jax.ShapeDtypeStruct((ns, 8), jnp.float32)]
    if fuse_next:
        out_specs += [row, row]
        out_shape += [jax.ShapeDtypeStruct((ns, FDIM), jnp.float32)] * 2
    if do_nh:
        out_specs += [pl.BlockSpec((1, 1), lambda i: (0, 0))]
        out_shape += [jax.ShapeDtypeStruct((1, 1), jnp.float32)]

    body = lambda *refs: _block_body(refs, fuse_next=fuse_next, do_nh=do_nh)
    args = [x, xi, xg, rbf] + wts + (nxt if fuse_next else [])
    return pl.pallas_call(
        body,
        grid=(grid,),
        in_specs=in_specs,
        out_specs=out_specs,
        out_shape=out_shape,
    )(*args)


def _block_weights(p):
    b = lambda v: v.reshape(1, FDIM)
    od = jnp.pad(p['odense'], ((0, 0), (0, 6)))
    k2f = jnp.pad(p['k2f'], ((0, 3), (0, 0)))
    rp = p['ires'][0]
    ap = p['ares'][0]
    op = p['ores'][0]
    return [k2f,
            rp['d1']['w'], b(rp['d1']['b']), rp['d2']['w'], b(rp['d2']['b']),
            p['dense']['w'], b(p['dense']['b']), b(p['u']),
            ap['d1']['w'], b(ap['d1']['b']), ap['d2']['w'], b(ap['d2']['b']),
            op['d1']['w'], b(op['d1']['b']), op['d2']['w'], b(op['d2']['b']),
            od, jnp.asarray(_PERM)]


def _run_block(x, xi, tj, rbf, idx_j, wts, nxt):
    N = x.shape[0]
    ns = N // NSLICE
    es = ns * GRP
    outs = []
    for s in range(NSLICE):
        xg = _sc_gather(tj, lax.slice(idx_j, (s * es,), ((s + 1) * es,)))
        outs.append(_block_call(x, xi, xg, rbf, wts, nxt, s * ns, ns))
    cat = lambda k: jnp.concatenate([o[k] for o in outs], axis=0)
    if nxt is not None:
        return cat(0), cat(1), cat(2), cat(3), None
    nh = sum(o[2][0, 0] for o in outs)
    return cat(0), cat(1), None, None, nh


def kernel(Z, R, idx_i, idx_j, params):
    N = Z.shape[0]
    Z = Z.astype(jnp.int32)
    idx_i = idx_i.astype(jnp.int32)
    idx_j = idx_j.astype(jnp.int32)

    d2 = _sc_edge_d2(R.astype(jnp.float32), idx_i, idx_j)
    rbf = _rbf_call(d2)

    b1, b2 = params['blocks']
    x, xi1, tj1 = _proj_call(Z, params['emb'],
                             b1['di']['w'], b1['di']['b'].reshape(1, FDIM),
                             b1['dj']['w'], b1['dj']['b'].reshape(1, FDIM))
    nxt = [b2['di']['w'], b2['di']['b'].reshape(1, FDIM),
           b2['dj']['w'], b2['dj']['b'].reshape(1, FDIM)]
    x1, out1, xi2, tj2, _ = _run_block(x, xi1, tj1, rbf, idx_j,
                                       _block_weights(b1), nxt)
    x2, out2, _, _, nh = _run_block(x1, xi2, tj2, rbf, idx_j,
                                    _block_weights(b2), None)

    e_total = out1[:, 0] + out2[:, 0]
    q_total = out1[:, 1] + out2[:, 1]
    nhloss = nh / np.float32(N * 2)
    return (e_total, q_total, nhloss)


# speedup vs baseline: 4.5581x; 1.0550x over previous
import functools

import numpy as np
import jax
import jax.numpy as jnp
from jax import lax
from jax.experimental import pallas as pl
from jax.experimental.pallas import tpu as pltpu
from jax.experimental.pallas import tpu_sc as plsc

FDIM = 128
KRBF = 5
CUTOFF = 10.0
GRP = 32
NSLICE = 5
BN = 400
BE = BN * GRP
SC_CH = 200
SC_NW = 32
LN2 = np.float32(np.log(2.0))
_CSTEP = float((np.exp(-CUTOFF) - 1.0) / (KRBF - 1))
_WIDTH = np.float32((0.5 / ((1.0 - np.exp(-CUTOFF)) / KRBF)) ** 2)


def _act(x):
    return jax.nn.softplus(x) - LN2


@functools.lru_cache(maxsize=None)
def _sc_gather_fn(V, D, B):
    per_w = B // SC_NW
    assert per_w % SC_CH == 0 and D % 128 == 0
    n_it = per_w // SC_CH
    mesh = plsc.VectorSubcoreMesh(core_axis_name="c", subcore_axis_name="s")

    @functools.partial(
        pl.kernel,
        mesh=mesh,
        out_type=jax.ShapeDtypeStruct((B, D), jnp.float32),
        scratch_types=[
            pltpu.VMEM((SC_CH,), jnp.int32),
            pltpu.VMEM((SC_CH,), jnp.int32),
            pltpu.VMEM((SC_CH, D), jnp.float32),
            pltpu.VMEM((SC_CH, D), jnp.float32),
            pltpu.SemaphoreType.DMA,
            pltpu.SemaphoreType.DMA,
        ],
    )
    def k(table_hbm, idx_hbm, out_hbm, i0, i1, r0, r1, s0, s1):
        wid = lax.axis_index("s") * 2 + lax.axis_index("c")
        base = wid * per_w
        slots = [(i0, r0, s0), (i1, r1, s1)]
        handles = [None, None]

        for c in range(n_it):
            iv, rv, sm = slots[c % 2]
            if handles[c % 2] is not None:
                handles[c % 2].wait()
                pltpu.sync_copy(
                    rv, out_hbm.at[pl.ds(base + (c - 2) * SC_CH, SC_CH)])
            pltpu.sync_copy(idx_hbm.at[pl.ds(base + c * SC_CH, SC_CH)], iv)
            handles[c % 2] = pltpu.async_copy(table_hbm.at[iv], rv, sm)
        for c in range(max(0, n_it - 2), n_it):
            iv, rv, sm = slots[c % 2]
            handles[c % 2].wait()
            pltpu.sync_copy(rv, out_hbm.at[pl.ds(base + c * SC_CH, SC_CH)])

    return k


def _sc_gather(table, idx):
    V, D = table.shape
    B = idx.shape[0]
    return _sc_gather_fn(V, D, B)(table, idx)


@functools.lru_cache(maxsize=None)
def _sc_d2_fn(N, E):
    assert E % (SC_NW * 16) == 0
    per_w = E // SC_NW
    nv = per_w // 16
    mesh = plsc.VectorSubcoreMesh(core_axis_name="c", subcore_axis_name="s")

    @functools.partial(
        pl.kernel,
        mesh=mesh,
        compiler_params=pltpu.CompilerParams(needs_layout_passes=False),
        out_type=jax.ShapeDtypeStruct((E,), jnp.float32),
        scratch_types=[
            pltpu.VMEM((N,), jnp.float32),
            pltpu.VMEM((N,), jnp.float32),
            pltpu.VMEM((N,), jnp.float32),
            pltpu.VMEM((per_w,), jnp.int32),
            pltpu.VMEM((per_w,), jnp.int32),
            pltpu.VMEM((per_w,), jnp.float32),
        ],
    )
    def k(rx_h, ry_h, rz_h, ii_h, ij_h, out_h, rx, ry, rz, iiv, ijv, d2v):
        wid = lax.axis_index("s") * 2 + lax.axis_index("c")
        base = wid * per_w
        pltpu.sync_copy(rx_h, rx)
        pltpu.sync_copy(ry_h, ry)
        pltpu.sync_copy(rz_h, rz)
        pltpu.sync_copy(ii_h.at[pl.ds(base, per_w)], iiv)
        pltpu.sync_copy(ij_h.at[pl.ds(base, per_w)], ijv)

        def body(v, carry):
            s = pl.ds(v * 16, 16)
            ii = iiv[s]
            ij = ijv[s]
            dx = plsc.load_gather(rx, [ii]) - plsc.load_gather(rx, [ij])
            dy = plsc.load_gather(ry, [ii]) - plsc.load_gather(ry, [ij])
            dz = plsc.load_gather(rz, [ii]) - plsc.load_gather(rz, [ij])
            d2v[s] = dx * dx + dy * dy + dz * dz
            return carry

        lax.fori_loop(0, nv, body, 0)
        pltpu.sync_copy(d2v, out_h.at[pl.ds(base, per_w)])

    return k


def _sc_edge_d2(R, idx_i, idx_j):
    N = R.shape[0]
    E = idx_i.shape[0]
    return _sc_d2_fn(N, E)(R[:, 0], R[:, 1], R[:, 2], idx_i, idx_j)


def _rbf_body(d2_ref, out_ref):
    d2 = d2_ref[...]
    D = jnp.sqrt(jnp.maximum(d2, 0.0))
    x = D / CUTOFF
    x3 = x ** 3
    x4 = x3 * x
    x5 = x4 * x
    cf = jnp.where(x < 1.0, 1.0 - 6.0 * x5 + 15.0 * x4 - 10.0 * x3,
                   jnp.zeros_like(x))
    eD = jnp.exp(-D)
    kk = lax.broadcasted_iota(jnp.int32, (8, 1, 1), 0)
    cen = jnp.where(kk < KRBF, 1.0 + kk.astype(jnp.float32) * _CSTEP, 0.0)
    msk = (kk < KRBF).astype(jnp.float32)
    val = cf[None] * jnp.exp(-_WIDTH * (eD[None] - cen) ** 2) * msk
    out_ref[...] = val.reshape(8, d2.size)


def _rbf_call(d2):
    E = d2.shape[0]
    return pl.pallas_call(
        _rbf_body,
        out_shape=jax.ShapeDtypeStruct((8, E), jnp.float32),
    )(d2.reshape(E // 128, 128))


def _proj_body(z_ref, emb_ref, wi_ref, bi_ref, wj_ref, bj_ref,
               x_ref, xi_ref, tj_ref):
    z = z_ref[...]
    oh = (z == lax.broadcasted_iota(jnp.int32, (z.shape[0], 32), 1))
    x = jnp.dot(oh.astype(jnp.float32), emb_ref[...],
                preferred_element_type=jnp.float32)
    xa = _act(x)
    x_ref[...] = x
    xi_ref[...] = (jnp.dot(xa, wi_ref[...], preferred_element_type=jnp.float32)
                   + bi_ref[...])
    tj_ref[...] = (jnp.dot(xa, wj_ref[...], preferred_element_type=jnp.float32)
                   + bj_ref[...])


def _proj_call(Z, emb, wi, bi, wj, bj):
    N = Z.shape[0]
    grid = N // BN
    full = lambda shape: pl.BlockSpec(shape, lambda i: tuple(0 for _ in shape))
    row = pl.BlockSpec((BN, FDIM), lambda i: (i, 0))
    embp = jnp.pad(emb, ((0, 32 - emb.shape[0]), (0, 0)))
    return pl.pallas_call(
        _proj_body,
        grid=(grid,),
        in_specs=[pl.BlockSpec((BN, 1), lambda i: (i, 0)), full((32, FDIM)),
                  full((FDIM, FDIM)), full((1, FDIM)),
                  full((FDIM, FDIM)), full((1, FDIM))],
        out_specs=[row, row, row],
        out_shape=[jax.ShapeDtypeStruct((N, FDIM), jnp.float32)] * 3,
    )(Z.reshape(N, 1), embp, wi, bi, wj, bj)


_PERM = np.zeros((FDIM, FDIM), np.float32)
for _r in range(4):
    for _q in range(GRP):
        _PERM[4 * _q + _r, 32 * _r + _q] = 1.0


def _block_body(refs, *, fuse_next, do_nh):
    (x_ref, xi_ref, xg_ref, rbf_ref, k2f_ref,
     i1w, i1b, i2w, i2b, dw, db, u_ref,
     a1w, a1b, a2w, a2b, o1w, o1b, o2w, o2b, od_ref, p_ref) = refs[:22]
    pos = 22
    if fuse_next:
        wi2, bi2, wj2, bj2 = refs[pos:pos + 4]
        pos += 4
    xout_ref = refs[pos]
    out_ref = refs[pos + 1]
    pos += 2
    if fuse_next:
        xi2_ref, tj2_ref = refs[pos:pos + 2]
        pos += 2
    if do_nh:
        nh_ref = refs[pos]

    dot = lambda a, b: jnp.dot(a, b, preferred_element_type=jnp.float32)

    g = lax.dot_general(rbf_ref[...], k2f_ref[...],
                        (((0,), (0,)), ((), ())),
                        preferred_element_type=jnp.float32)
    xj = g * xg_ref[...]
    X = xj.reshape(BN, GRP, FDIM)
    xi = xi_ref[...]
    xiP = dot(xi, p_ref[...])

    att = jnp.zeros((BN, GRP), jnp.float32)
    for r in range(4):
        xs = X[:, :, 32 * r:32 * r + 32]
        xir = xiP[:, 32 * r:32 * r + 32].reshape(BN, GRP, 1)
        att = att + jnp.sum(xs * xir, axis=1)
    att = att - jnp.max(att, axis=1, keepdims=True)
    ea = jnp.exp(att)
    attw = ea / jnp.sum(ea, axis=1, keepdims=True)
    xjagg = jnp.sum(X * attw.reshape(BN, GRP, 1), axis=1)

    m = xi + xjagg
    m = m + dot(dot(_act(m), i1w[...]) + i1b[...], i2w[...]) + i2b[...]
    m = _act(m)
    xnew = u_ref[...] * x_ref[...] + dot(m, dw[...]) + db[...]
    xo = xnew + dot(dot(_act(xnew), a1w[...]) + a1b[...], a2w[...]) + a2b[...]
    h = xo + dot(dot(_act(xo), o1w[...]) + o1b[...], o2w[...]) + o2b[...]
    h = _act(h)
    out = dot(h, od_ref[...])

    xout_ref[...] = xo
    out_ref[...] = out
    if fuse_next:
        xa2 = _act(xo)
        xi2_ref[...] = dot(xa2, wi2[...]) + bi2[...]
        tj2_ref[...] = dot(xa2, wj2[...]) + bj2[...]
    if do_nh:
        o2 = out * out
        part = jnp.sum(o2 / (o2 + 1e-7)).reshape(1, 1)

        @pl.when(pl.program_id(0) == 0)
        def _():
            nh_ref[...] = jnp.zeros_like(nh_ref)

        nh_ref[...] += part


def _block_call(x, xi, xg, rbf, wts, nxt, node0, ns):
    grid = ns // BN
    ro = node0 // BN
    fuse_next = nxt is not None
    do_nh = not fuse_next

    full = lambda shape: pl.BlockSpec(shape, lambda i: tuple(0 for _ in shape))
    rowo = pl.BlockSpec((BN, FDIM), lambda i: (i + ro, 0))
    row = pl.BlockSpec((BN, FDIM), lambda i: (i, 0))
    erow = pl.BlockSpec((BE, FDIM), lambda i: (i, 0))
    rrow = pl.BlockSpec((8, BE), lambda i: (0, i + ro))
    orow = pl.BlockSpec((BN, 8), lambda i: (i, 0))

    in_specs = [rowo, rowo, erow, rrow, full((8, FDIM))]
    in_specs += [full((FDIM, FDIM)), full((1, FDIM))] * 2
    in_specs += [full((FDIM, FDIM)), full((1, FDIM)), full((1, FDIM))]
    in_specs += [full((FDIM, FDIM)), full((1, FDIM))] * 4
    in_specs += [full((FDIM, 8)), full((FDIM, FDIM))]
    if fuse_next:
        in_specs += [full((FDIM, FDIM)), full((1, FDIM))] * 2

    out_specs = [row, orow]
    out_shape = [jax.ShapeDtypeStruct((ns, FDIM), jnp.float32),
                 jax.ShapeDtypeStruct((ns, 8), jnp.float32)]
    if fuse_next:
        out_specs += [row, row]
        out_shape += [jax.ShapeDtypeStruct((ns, FDIM), jnp.float32)] * 2
    if do_nh:
        out_specs += [pl.BlockSpec((1, 1), lambda i: (0, 0))]
        out_shape += [jax.ShapeDtypeStruct((1, 1), jnp.float32)]

    body = lambda *refs: _block_body(refs, fuse_next=fuse_next, do_nh=do_nh)
    args = [x, xi, xg, rbf] + wts + (nxt if fuse_next else [])
    return pl.pallas_call(
        body,
        grid=(grid,),
        in_specs=in_specs,
        out_specs=out_specs,
        out_shape=out_shape,
    )(*args)


def _block_weights(p):
    b = lambda v: v.reshape(1, FDIM)
    od = jnp.pad(p['odense'], ((0, 0), (0, 6)))
    k2f = jnp.pad(p['k2f'], ((0, 3), (0, 0)))
    rp = p['ires'][0]
    ap = p['ares'][0]
    op = p['ores'][0]
    return [k2f,
            rp['d1']['w'], b(rp['d1']['b']), rp['d2']['w'], b(rp['d2']['b']),
            p['dense']['w'], b(p['dense']['b']), b(p['u']),
            ap['d1']['w'], b(ap['d1']['b']), ap['d2']['w'], b(ap['d2']['b']),
            op['d1']['w'], b(op['d1']['b']), op['d2']['w'], b(op['d2']['b']),
            od, jnp.asarray(_PERM)]


def _run_block(x, xi, tj, rbf, idx_j, wts, nxt):
    N = x.shape[0]
    ns = N // NSLICE
    es = ns * GRP
    outs = []
    for s in range(NSLICE):
        xg = _sc_gather(tj, lax.slice(idx_j, (s * es,), ((s + 1) * es,)))
        outs.append(_block_call(x, xi, xg, rbf, wts, nxt, s * ns, ns))
    cat = lambda k: jnp.concatenate([o[k] for o in outs], axis=0)
    if nxt is not None:
        return cat(0), cat(1), cat(2), cat(3), None
    nh = sum(o[2][0, 0] for o in outs)
    return cat(0), cat(1), None, None, nh


def kernel(Z, R, idx_i, idx_j, params):
    N = Z.shape[0]
    Z = Z.astype(jnp.int32)
    idx_i = idx_i.astype(jnp.int32)
    idx_j = idx_j.astype(jnp.int32)

    d2 = _sc_edge_d2(R.astype(jnp.float32), idx_i, idx_j)
    rbf = _rbf_call(d2)

    b1, b2 = params['blocks']
    x, xi1, tj1 = _proj_call(Z, params['emb'],
                             b1['di']['w'], b1['di']['b'].reshape(1, FDIM),
                             b1['dj']['w'], b1['dj']['b'].reshape(1, FDIM))
    nxt = [b2['di']['w'], b2['di']['b'].reshape(1, FDIM),
           b2['dj']['w'], b2['dj']['b'].reshape(1, FDIM)]
    x1, out1, xi2, tj2, _ = _run_block(x, xi1, tj1, rbf, idx_j,
                                       _block_weights(b1), nxt)
    x2, out2, _, _, nh = _run_block(x1, xi2, tj2, rbf, idx_j,
                                    _block_weights(b2), None)

    e_total = out1[:, 0] + out2[:, 0]
    q_total = out1[:, 1] + out2[:, 1]
    nhloss = nh / np.float32(N * 2)
    return (e_total, q_total, nhloss)


# speedup vs baseline: 5.3549x; 1.1748x over previous
import functools

import numpy as np
import jax
import jax.numpy as jnp
from jax import lax
from jax.experimental import pallas as pl
from jax.experimental.pallas import tpu as pltpu
from jax.experimental.pallas import tpu_sc as plsc

FDIM = 128
KRBF = 5
CUTOFF = 10.0
GRP = 32
NSLICE = 5
BN = 400
BE = BN * GRP
SC_CH = 200
SC_NW = 32
LN2 = np.float32(np.log(2.0))
_CSTEP = float((np.exp(-CUTOFF) - 1.0) / (KRBF - 1))
_WIDTH = np.float32((0.5 / ((1.0 - np.exp(-CUTOFF)) / KRBF)) ** 2)


def _act(x):
    return jax.nn.softplus(x) - LN2


@functools.lru_cache(maxsize=None)
def _sc_gather_fn(V, D, B):
    per_w = B // SC_NW
    assert per_w % SC_CH == 0 and D % 128 == 0
    n_it = per_w // SC_CH
    mesh = plsc.VectorSubcoreMesh(core_axis_name="c", subcore_axis_name="s")

    @functools.partial(
        pl.kernel,
        mesh=mesh,
        out_type=jax.ShapeDtypeStruct((B, D), jnp.float32),
        scratch_types=[
            pltpu.VMEM((SC_CH,), jnp.int32),
            pltpu.VMEM((SC_CH,), jnp.int32),
            pltpu.VMEM((SC_CH, D), jnp.float32),
            pltpu.VMEM((SC_CH, D), jnp.float32),
            pltpu.SemaphoreType.DMA,
            pltpu.SemaphoreType.DMA,
        ],
    )
    def k(table_hbm, idx_hbm, out_hbm, i0, i1, r0, r1, s0, s1):
        wid = lax.axis_index("s") * 2 + lax.axis_index("c")
        base = wid * per_w
        slots = [(i0, r0, s0), (i1, r1, s1)]
        handles = [None, None]

        for c in range(n_it):
            iv, rv, sm = slots[c % 2]
            if handles[c % 2] is not None:
                handles[c % 2].wait()
                pltpu.sync_copy(
                    rv, out_hbm.at[pl.ds(base + (c - 2) * SC_CH, SC_CH)])
            pltpu.sync_copy(idx_hbm.at[pl.ds(base + c * SC_CH, SC_CH)], iv)
            handles[c % 2] = pltpu.async_copy(table_hbm.at[iv], rv, sm)
        for c in range(max(0, n_it - 2), n_it):
            iv, rv, sm = slots[c % 2]
            handles[c % 2].wait()
            pltpu.sync_copy(rv, out_hbm.at[pl.ds(base + c * SC_CH, SC_CH)])

    return k


def _sc_gather(table, idx):
    V, D = table.shape
    B = idx.shape[0]
    return _sc_gather_fn(V, D, B)(table, idx)


@functools.lru_cache(maxsize=None)
def _sc_d2_fn(N, E):
    assert E % (SC_NW * 16) == 0
    per_w = E // SC_NW
    nv = per_w // 16
    mesh = plsc.VectorSubcoreMesh(core_axis_name="c", subcore_axis_name="s")

    @functools.partial(
        pl.kernel,
        mesh=mesh,
        compiler_params=pltpu.CompilerParams(needs_layout_passes=False),
        out_type=jax.ShapeDtypeStruct((E,), jnp.float32),
        scratch_types=[
            pltpu.VMEM((N,), jnp.float32),
            pltpu.VMEM((N,), jnp.float32),
            pltpu.VMEM((N,), jnp.float32),
            pltpu.VMEM((per_w,), jnp.int32),
            pltpu.VMEM((per_w,), jnp.int32),
            pltpu.VMEM((per_w,), jnp.float32),
        ],
    )
    def k(rx_h, ry_h, rz_h, ii_h, ij_h, out_h, rx, ry, rz, iiv, ijv, d2v):
        wid = lax.axis_index("s") * 2 + lax.axis_index("c")
        base = wid * per_w
        pltpu.sync_copy(rx_h, rx)
        pltpu.sync_copy(ry_h, ry)
        pltpu.sync_copy(rz_h, rz)
        pltpu.sync_copy(ii_h.at[pl.ds(base, per_w)], iiv)
        pltpu.sync_copy(ij_h.at[pl.ds(base, per_w)], ijv)

        def body(v, carry):
            s = pl.ds(v * 16, 16)
            ii = iiv[s]
            ij = ijv[s]
            dx = plsc.load_gather(rx, [ii]) - plsc.load_gather(rx, [ij])
            dy = plsc.load_gather(ry, [ii]) - plsc.load_gather(ry, [ij])
            dz = plsc.load_gather(rz, [ii]) - plsc.load_gather(rz, [ij])
            d2v[s] = dx * dx + dy * dy + dz * dz
            return carry

        lax.fori_loop(0, nv, body, 0)
        pltpu.sync_copy(d2v, out_h.at[pl.ds(base, per_w)])

    return k


def _sc_edge_d2(R, idx_i, idx_j):
    N = R.shape[0]
    E = idx_i.shape[0]
    return _sc_d2_fn(N, E)(R[:, 0], R[:, 1], R[:, 2], idx_i, idx_j)


def _rbf_body(d2_ref, out_ref):
    d2 = d2_ref[...]
    D = jnp.sqrt(jnp.maximum(d2, 0.0))
    x = D / CUTOFF
    x3 = x ** 3
    x4 = x3 * x
    x5 = x4 * x
    cf = jnp.where(x < 1.0, 1.0 - 6.0 * x5 + 15.0 * x4 - 10.0 * x3,
                   jnp.zeros_like(x))
    eD = jnp.exp(-D)
    kk = lax.broadcasted_iota(jnp.int32, (8, 1, 1), 0)
    cen = jnp.where(kk < KRBF, 1.0 + kk.astype(jnp.float32) * _CSTEP, 0.0)
    msk = (kk < KRBF).astype(jnp.float32)
    val = cf[None] * jnp.exp(-_WIDTH * (eD[None] - cen) ** 2) * msk
    out_ref[...] = val.reshape(8, d2.size)


def _rbf_call(d2):
    E = d2.shape[0]
    return pl.pallas_call(
        _rbf_body,
        out_shape=jax.ShapeDtypeStruct((8, E), jnp.float32),
    )(d2.reshape(E // 128, 128))


def _proj_body(z_ref, emb_ref, wi_ref, bi_ref, wj_ref, bj_ref,
               x_ref, xi_ref, tj_ref):
    z = z_ref[...]
    oh = (z == lax.broadcasted_iota(jnp.int32, (z.shape[0], 32), 1))
    x = jnp.dot(oh.astype(jnp.float32), emb_ref[...],
                preferred_element_type=jnp.float32)
    xa = _act(x)
    x_ref[...] = x
    xi_ref[...] = (jnp.dot(xa, wi_ref[...], preferred_element_type=jnp.float32)
                   + bi_ref[...])
    tj_ref[...] = (jnp.dot(xa, wj_ref[...], preferred_element_type=jnp.float32)
                   + bj_ref[...])


def _proj_call(Z, emb, wi, bi, wj, bj):
    N = Z.shape[0]
    grid = N // BN
    full = lambda shape: pl.BlockSpec(shape, lambda i: tuple(0 for _ in shape))
    row = pl.BlockSpec((BN, FDIM), lambda i: (i, 0))
    embp = jnp.pad(emb, ((0, 32 - emb.shape[0]), (0, 0)))
    return pl.pallas_call(
        _proj_body,
        grid=(grid,),
        in_specs=[pl.BlockSpec((BN, 1), lambda i: (i, 0)), full((32, FDIM)),
                  full((FDIM, FDIM)), full((1, FDIM)),
                  full((FDIM, FDIM)), full((1, FDIM))],
        out_specs=[row, row, row],
        out_shape=[jax.ShapeDtypeStruct((N, FDIM), jnp.float32)] * 3,
    )(Z.reshape(N, 1), embp, wi, bi, wj, bj)


_ff = np.arange(FDIM)
_qq = np.arange(GRP)
_MB = np.ascontiguousarray(
    (_ff[:, None, None] == (4 * _qq[:, None] + _ff[None, :] // 32)[None])
    .reshape(FDIM, GRP * FDIM), np.float32)
_H = np.ascontiguousarray(
    np.broadcast_to((_ff % 32)[:, None] == (_ff % 32)[None, :],
                    (GRP, FDIM, FDIM)).reshape(GRP * FDIM, FDIM), np.float32)
_MA = np.ascontiguousarray(
    np.broadcast_to((_ff[:, None] == _qq[None, :])[:, :, None],
                    (FDIM, GRP, FDIM)).reshape(FDIM, GRP * FDIM), np.float32)
_K = np.ascontiguousarray(
    np.broadcast_to(np.eye(FDIM, dtype=np.float32),
                    (GRP, FDIM, FDIM)).reshape(GRP * FDIM, FDIM), np.float32)


def _block_body(refs, *, fuse_next, do_nh):
    (x_ref, xi_ref, xg_ref, rbf_ref, k2f_ref,
     i1w, i1b, i2w, i2b, dw, db, u_ref,
     a1w, a1b, a2w, a2b, o1w, o1b, o2w, o2b, od_ref,
     mb_ref, h_ref, ma_ref, kk_ref) = refs[:25]
    pos = 25
    if fuse_next:
        wi2, bi2, wj2, bj2 = refs[pos:pos + 4]
        pos += 4
    xout_ref = refs[pos]
    out_ref = refs[pos + 1]
    pos += 2
    if fuse_next:
        xi2_ref, tj2_ref = refs[pos:pos + 2]
        pos += 2
    if do_nh:
        nh_ref = refs[pos]

    dot = lambda a, b: jnp.dot(a, b, preferred_element_type=jnp.float32)

    g = lax.dot_general(rbf_ref[...], k2f_ref[...],
                        (((0,), (0,)), ((), ())),
                        preferred_element_type=jnp.float32)
    xj = g * xg_ref[...]
    xi = xi_ref[...]

    B = dot(xi, mb_ref[...]).reshape(BE, FDIM)
    att = dot((xj * B).reshape(BN, GRP * FDIM), h_ref[...])
    att = att - jnp.max(att, axis=1, keepdims=True)
    ea = jnp.exp(att)
    attw = ea / (jnp.sum(ea, axis=1, keepdims=True) * 0.25)
    attwE = dot(attw, ma_ref[...]).reshape(BE, FDIM)
    xjagg = dot((xj * attwE).reshape(BN, GRP * FDIM), kk_ref[...])

    m = xi + xjagg
    m = m + dot(dot(_act(m), i1w[...]) + i1b[...], i2w[...]) + i2b[...]
    m = _act(m)
    xnew = u_ref[...] * x_ref[...] + dot(m, dw[...]) + db[...]
    xo = xnew + dot(dot(_act(xnew), a1w[...]) + a1b[...], a2w[...]) + a2b[...]
    h = xo + dot(dot(_act(xo), o1w[...]) + o1b[...], o2w[...]) + o2b[...]
    h = _act(h)
    out = dot(h, od_ref[...])

    xout_ref[...] = xo
    out_ref[...] = out
    if fuse_next:
        xa2 = _act(xo)
        xi2_ref[...] = dot(xa2, wi2[...]) + bi2[...]
        tj2_ref[...] = dot(xa2, wj2[...]) + bj2[...]
    if do_nh:
        o2 = out * out
        part = jnp.sum(o2 / (o2 + 1e-7)).reshape(1, 1)

        @pl.when(pl.program_id(0) == 0)
        def _():
            nh_ref[...] = jnp.zeros_like(nh_ref)

        nh_ref[...] += part


def _block_call(x, xi, xg, rbf, wts, nxt, node0, ns):
    grid = ns // BN
    ro = node0 // BN
    fuse_next = nxt is not None
    do_nh = not fuse_next

    full = lambda shape: pl.BlockSpec(shape, lambda i: tuple(0 for _ in shape))
    rowo = pl.BlockSpec((BN, FDIM), lambda i: (i + ro, 0))
    row = pl.BlockSpec((BN, FDIM), lambda i: (i, 0))
    erow = pl.BlockSpec((BE, FDIM), lambda i: (i, 0))
    rrow = pl.BlockSpec((8, BE), lambda i: (0, i + ro))
    orow = pl.BlockSpec((BN, 8), lambda i: (i, 0))

    in_specs = [rowo, rowo, erow, rrow, full((8, FDIM))]
    in_specs += [full((FDIM, FDIM)), full((1, FDIM))] * 2
    in_specs += [full((FDIM, FDIM)), full((1, FDIM)), full((1, FDIM))]
    in_specs += [full((FDIM, FDIM)), full((1, FDIM))] * 4
    in_specs += [full((FDIM, 8))]
    in_specs += [full((FDIM, GRP * FDIM)), full((GRP * FDIM, FDIM)),
                 full((FDIM, GRP * FDIM)), full((GRP * FDIM, FDIM))]
    if fuse_next:
        in_specs += [full((FDIM, FDIM)), full((1, FDIM))] * 2

    out_specs = [row, orow]
    out_shape = [jax.ShapeDtypeStruct((ns, FDIM), jnp.float32),
                 jax.ShapeDtypeStruct((ns, 8), jnp.float32)]
    if fuse_next:
        out_specs += [row, row]
        out_shape += [jax.ShapeDtypeStruct((ns, FDIM), jnp.float32)] * 2
    if do_nh:
        out_specs += [pl.BlockSpec((1, 1), lambda i: (0, 0))]
        out_shape += [jax.ShapeDtypeStruct((1, 1), jnp.float32)]

    body = lambda *refs: _block_body(refs, fuse_next=fuse_next, do_nh=do_nh)
    args = [x, xi, xg, rbf] + wts + (nxt if fuse_next else [])
    return pl.pallas_call(
        body,
        grid=(grid,),
        in_specs=in_specs,
        out_specs=out_specs,
        out_shape=out_shape,
    )(*args)


def _block_weights(p):
    b = lambda v: v.reshape(1, FDIM)
    od = jnp.pad(p['odense'], ((0, 0), (0, 6)))
    k2f = jnp.pad(p['k2f'], ((0, 3), (0, 0)))
    rp = p['ires'][0]
    ap = p['ares'][0]
    op = p['ores'][0]
    return [k2f,
            rp['d1']['w'], b(rp['d1']['b']), rp['d2']['w'], b(rp['d2']['b']),
            p['dense']['w'], b(p['dense']['b']), b(p['u']),
            ap['d1']['w'], b(ap['d1']['b']), ap['d2']['w'], b(ap['d2']['b']),
            op['d1']['w'], b(op['d1']['b']), op['d2']['w'], b(op['d2']['b']),
            od, jnp.asarray(_MB), jnp.asarray(_H),
            jnp.asarray(_MA), jnp.asarray(_K)]


def _run_block(x, xi, tj, rbf, idx_j, wts, nxt):
    N = x.shape[0]
    ns = N // NSLICE
    es = ns * GRP
    outs = []
    for s in range(NSLICE):
        xg = _sc_gather(tj, lax.slice(idx_j, (s * es,), ((s + 1) * es,)))
        outs.append(_block_call(x, xi, xg, rbf, wts, nxt, s * ns, ns))
    cat = lambda k: jnp.concatenate([o[k] for o in outs], axis=0)
    if nxt is not None:
        return cat(0), cat(1), cat(2), cat(3), None
    nh = sum(o[2][0, 0] for o in outs)
    return cat(0), cat(1), None, None, nh


def kernel(Z, R, idx_i, idx_j, params):
    N = Z.shape[0]
    Z = Z.astype(jnp.int32)
    idx_i = idx_i.astype(jnp.int32)
    idx_j = idx_j.astype(jnp.int32)

    d2 = _sc_edge_d2(R.astype(jnp.float32), idx_i, idx_j)
    rbf = _rbf_call(d2)

    b1, b2 = params['blocks']
    x, xi1, tj1 = _proj_call(Z, params['emb'],
                             b1['di']['w'], b1['di']['b'].reshape(1, FDIM),
                             b1['dj']['w'], b1['dj']['b'].reshape(1, FDIM))
    nxt = [b2['di']['w'], b2['di']['b'].reshape(1, FDIM),
           b2['dj']['w'], b2['dj']['b'].reshape(1, FDIM)]
    x1, out1, xi2, tj2, _ = _run_block(x, xi1, tj1, rbf, idx_j,
                                       _block_weights(b1), nxt)
    x2, out2, _, _, nh = _run_block(x1, xi2, tj2, rbf, idx_j,
                                    _block_weights(b2), None)

    e_total = out1[:, 0] + out2[:, 0]
    q_total = out1[:, 1] + out2[:, 1]
    nhloss = nh / np.float32(N * 2)
    return (e_total, q_total, nhloss)


# speedup vs baseline: 6.4653x; 1.2074x over previous
import functools

import numpy as np
import jax
import jax.numpy as jnp
from jax import lax
from jax.experimental import pallas as pl
from jax.experimental.pallas import tpu as pltpu
from jax.experimental.pallas import tpu_sc as plsc

FDIM = 128
KRBF = 5
CUTOFF = 10.0
GRP = 32
NSLICE = 5
BN = 400
BE = BN * GRP
SC_CH = 200
SC_NW = 32
LN2 = np.float32(np.log(2.0))
_CSTEP = float((np.exp(-CUTOFF) - 1.0) / (KRBF - 1))
_WIDTH = np.float32((0.5 / ((1.0 - np.exp(-CUTOFF)) / KRBF)) ** 2)


def _act(x):
    return jax.nn.softplus(x) - LN2


@functools.lru_cache(maxsize=None)
def _sc_gather_fn(V, D, B):
    per_w = B // SC_NW
    assert per_w % SC_CH == 0 and D % 128 == 0
    n_it = per_w // SC_CH
    mesh = plsc.VectorSubcoreMesh(core_axis_name="c", subcore_axis_name="s")

    @functools.partial(
        pl.kernel,
        mesh=mesh,
        out_type=jax.ShapeDtypeStruct((B, D), jnp.float32),
        scratch_types=[
            pltpu.VMEM((SC_CH,), jnp.int32),
            pltpu.VMEM((SC_CH,), jnp.int32),
            pltpu.VMEM((SC_CH, D), jnp.float32),
            pltpu.VMEM((SC_CH, D), jnp.float32),
            pltpu.SemaphoreType.DMA,
            pltpu.SemaphoreType.DMA,
        ],
    )
    def k(table_hbm, idx_hbm, out_hbm, i0, i1, r0, r1, s0, s1):
        wid = lax.axis_index("s") * 2 + lax.axis_index("c")
        base = wid * per_w
        slots = [(i0, r0, s0), (i1, r1, s1)]
        handles = [None, None]

        for c in range(n_it):
            iv, rv, sm = slots[c % 2]
            if handles[c % 2] is not None:
                handles[c % 2].wait()
                pltpu.sync_copy(
                    rv, out_hbm.at[pl.ds(base + (c - 2) * SC_CH, SC_CH)])
            pltpu.sync_copy(idx_hbm.at[pl.ds(base + c * SC_CH, SC_CH)], iv)
            handles[c % 2] = pltpu.async_copy(table_hbm.at[iv], rv, sm)
        for c in range(max(0, n_it - 2), n_it):
            iv, rv, sm = slots[c % 2]
            handles[c % 2].wait()
            pltpu.sync_copy(rv, out_hbm.at[pl.ds(base + c * SC_CH, SC_CH)])

    return k


def _sc_gather(table, idx):
    V, D = table.shape
    B = idx.shape[0]
    return _sc_gather_fn(V, D, B)(table, idx)


@functools.lru_cache(maxsize=None)
def _sc_d2_fn(N, E):
    assert E % (SC_NW * 16) == 0
    per_w = E // SC_NW
    nv = per_w // 16
    mesh = plsc.VectorSubcoreMesh(core_axis_name="c", subcore_axis_name="s")

    @functools.partial(
        pl.kernel,
        mesh=mesh,
        compiler_params=pltpu.CompilerParams(needs_layout_passes=False),
        out_type=jax.ShapeDtypeStruct((E,), jnp.float32),
        scratch_types=[
            pltpu.VMEM((N,), jnp.float32),
            pltpu.VMEM((N,), jnp.float32),
            pltpu.VMEM((N,), jnp.float32),
            pltpu.VMEM((per_w,), jnp.int32),
            pltpu.VMEM((per_w,), jnp.int32),
            pltpu.VMEM((per_w,), jnp.float32),
        ],
    )
    def k(rx_h, ry_h, rz_h, ii_h, ij_h, out_h, rx, ry, rz, iiv, ijv, d2v):
        wid = lax.axis_index("s") * 2 + lax.axis_index("c")
        base = wid * per_w
        pltpu.sync_copy(rx_h, rx)
        pltpu.sync_copy(ry_h, ry)
        pltpu.sync_copy(rz_h, rz)
        pltpu.sync_copy(ii_h.at[pl.ds(base, per_w)], iiv)
        pltpu.sync_copy(ij_h.at[pl.ds(base, per_w)], ijv)

        def body(v, carry):
            s = pl.ds(v * 16, 16)
            ii = iiv[s]
            ij = ijv[s]
            dx = plsc.load_gather(rx, [ii]) - plsc.load_gather(rx, [ij])
            dy = plsc.load_gather(ry, [ii]) - plsc.load_gather(ry, [ij])
            dz = plsc.load_gather(rz, [ii]) - plsc.load_gather(rz, [ij])
            d2v[s] = dx * dx + dy * dy + dz * dz
            return carry

        lax.fori_loop(0, nv, body, 0)
        pltpu.sync_copy(d2v, out_h.at[pl.ds(base, per_w)])

    return k


def _sc_edge_d2(R, idx_i, idx_j):
    N = R.shape[0]
    E = idx_i.shape[0]
    return _sc_d2_fn(N, E)(R[:, 0], R[:, 1], R[:, 2], idx_i, idx_j)


def _rbf_body(d2_ref, out_ref):
    d2 = d2_ref[...]
    D = jnp.sqrt(jnp.maximum(d2, 0.0))
    x = D / CUTOFF
    x3 = x ** 3
    x4 = x3 * x
    x5 = x4 * x
    cf = jnp.where(x < 1.0, 1.0 - 6.0 * x5 + 15.0 * x4 - 10.0 * x3,
                   jnp.zeros_like(x))
    eD = jnp.exp(-D)
    kk = lax.broadcasted_iota(jnp.int32, (8, 1, 1), 0)
    cen = jnp.where(kk < KRBF, 1.0 + kk.astype(jnp.float32) * _CSTEP, 0.0)
    msk = (kk < KRBF).astype(jnp.float32)
    val = cf[None] * jnp.exp(-_WIDTH * (eD[None] - cen) ** 2) * msk
    out_ref[...] = val.reshape(8, d2.size)


def _rbf_call(d2):
    E = d2.shape[0]
    return pl.pallas_call(
        _rbf_body,
        out_shape=jax.ShapeDtypeStruct((8, E), jnp.float32),
    )(d2.reshape(E // 128, 128))


def _proj_body(z_ref, emb_ref, wi_ref, bi_ref, wj_ref, bj_ref,
               x_ref, xi_ref, tj_ref):
    z = z_ref[...]
    oh = (z == lax.broadcasted_iota(jnp.int32, (z.shape[0], 32), 1))
    x = jnp.dot(oh.astype(jnp.float32), emb_ref[...],
                preferred_element_type=jnp.float32)
    xa = _act(x)
    x_ref[...] = x
    xi_ref[...] = (jnp.dot(xa, wi_ref[...], preferred_element_type=jnp.float32)
                   + bi_ref[...])
    tj_ref[...] = (jnp.dot(xa, wj_ref[...], preferred_element_type=jnp.float32)
                   + bj_ref[...])


def _proj_call(Z, emb, wi, bi, wj, bj):
    N = Z.shape[0]
    grid = N // BN
    full = lambda shape: pl.BlockSpec(shape, lambda i: tuple(0 for _ in shape))
    row = pl.BlockSpec((BN, FDIM), lambda i: (i, 0))
    embp = jnp.pad(emb, ((0, 32 - emb.shape[0]), (0, 0)))
    return pl.pallas_call(
        _proj_body,
        grid=(grid,),
        in_specs=[pl.BlockSpec((BN, 1), lambda i: (i, 0)), full((32, FDIM)),
                  full((FDIM, FDIM)), full((1, FDIM)),
                  full((FDIM, FDIM)), full((1, FDIM))],
        out_specs=[row, row, row],
        out_shape=[jax.ShapeDtypeStruct((N, FDIM), jnp.float32)] * 3,
    )(Z.reshape(N, 1), embp, wi, bi, wj, bj)


_ff = np.arange(FDIM)
_qq = np.arange(GRP)
_MB = np.ascontiguousarray(
    (_ff[:, None, None] == (4 * _qq[:, None] + _ff[None, :] // 32)[None])
    .reshape(FDIM, GRP * FDIM), np.float32)
_H = np.ascontiguousarray(
    np.broadcast_to((_ff % 32)[:, None] == (_ff % 32)[None, :],
                    (GRP, FDIM, FDIM)).reshape(GRP * FDIM, FDIM), np.float32)
_MA = np.ascontiguousarray(
    np.broadcast_to((_ff[:, None] == _qq[None, :])[:, :, None],
                    (FDIM, GRP, FDIM)).reshape(FDIM, GRP * FDIM), np.float32)
_K = np.ascontiguousarray(
    np.broadcast_to(np.eye(FDIM, dtype=np.float32),
                    (GRP, FDIM, FDIM)).reshape(GRP * FDIM, FDIM), np.float32)


def _block_body(refs, *, fuse_next, do_nh):
    (x_ref, xi_ref, xg_ref, rbf_ref, k2f_ref,
     i1w, i1b, i2w, i2b, dw, db, u_ref,
     a1w, a1b, a2w, a2b, o1w, o1b, o2w, o2b, od_ref,
     mb_ref, h_ref) = refs[:23]
    pos = 23
    if fuse_next:
        wi2, bi2, wj2, bj2 = refs[pos:pos + 4]
        pos += 4
    xout_ref = refs[pos]
    out_ref = refs[pos + 1]
    pos += 2
    if fuse_next:
        xi2_ref, tj2_ref = refs[pos:pos + 2]
        pos += 2
    if do_nh:
        nh_ref = refs[pos]

    dot = lambda a, b: jnp.dot(a, b, preferred_element_type=jnp.float32)

    g = lax.dot_general(rbf_ref[...], k2f_ref[...],
                        (((0,), (0,)), ((), ())),
                        preferred_element_type=jnp.float32)
    xj = g * xg_ref[...]
    xi = xi_ref[...]

    B = dot(xi, mb_ref[...]).reshape(BE, FDIM)
    att = dot((xj * B).reshape(BN, GRP * FDIM), h_ref[...])
    att = att - jnp.max(att, axis=1, keepdims=True)
    ea = jnp.exp(att)
    attw = ea / (jnp.sum(ea, axis=1, keepdims=True) * 0.25)
    xjagg = lax.dot_general(attw[:, None, :GRP], xj.reshape(BN, GRP, FDIM),
                            (((2,), (1,)), ((0,), (0,))),
                            preferred_element_type=jnp.float32)[:, 0]

    m = xi + xjagg
    m = m + dot(dot(_act(m), i1w[...]) + i1b[...], i2w[...]) + i2b[...]
    m = _act(m)
    xnew = u_ref[...] * x_ref[...] + dot(m, dw[...]) + db[...]
    xo = xnew + dot(dot(_act(xnew), a1w[...]) + a1b[...], a2w[...]) + a2b[...]
    h = xo + dot(dot(_act(xo), o1w[...]) + o1b[...], o2w[...]) + o2b[...]
    h = _act(h)
    out = dot(h, od_ref[...])

    xout_ref[...] = xo
    out_ref[...] = out
    if fuse_next:
        xa2 = _act(xo)
        xi2_ref[...] = dot(xa2, wi2[...]) + bi2[...]
        tj2_ref[...] = dot(xa2, wj2[...]) + bj2[...]
    if do_nh:
        o2 = out * out
        part = jnp.sum(o2 / (o2 + 1e-7)).reshape(1, 1)

        @pl.when(pl.program_id(0) == 0)
        def _():
            nh_ref[...] = jnp.zeros_like(nh_ref)

        nh_ref[...] += part


def _block_call(x, xi, xg, rbf, wts, nxt, node0, ns):
    grid = ns // BN
    ro = node0 // BN
    fuse_next = nxt is not None
    do_nh = not fuse_next

    full = lambda shape: pl.BlockSpec(shape, lambda i: tuple(0 for _ in shape))
    rowo = pl.BlockSpec((BN, FDIM), lambda i: (i + ro, 0))
    row = pl.BlockSpec((BN, FDIM), lambda i: (i, 0))
    erow = pl.BlockSpec((BE, FDIM), lambda i: (i, 0))
    rrow = pl.BlockSpec((8, BE), lambda i: (0, i + ro))
    orow = pl.BlockSpec((BN, 8), lambda i: (i, 0))

    in_specs = [rowo, rowo, erow, rrow, full((8, FDIM))]
    in_specs += [full((FDIM, FDIM)), full((1, FDIM))] * 2
    in_specs += [full((FDIM, FDIM)), full((1, FDIM)), full((1, FDIM))]
    in_specs += [full((FDIM, FDIM)), full((1, FDIM))] * 4
    in_specs += [full((FDIM, 8))]
    in_specs += [full((FDIM, GRP * FDIM)), full((GRP * FDIM, FDIM))]
    if fuse_next:
        in_specs += [full((FDIM, FDIM)), full((1, FDIM))] * 2

    out_specs = [row, orow]
    out_shape = [jax.ShapeDtypeStruct((ns, FDIM), jnp.float32),
                 jax.ShapeDtypeStruct((ns, 8), jnp.float32)]
    if fuse_next:
        out_specs += [row, row]
        out_shape += [jax.ShapeDtypeStruct((ns, FDIM), jnp.float32)] * 2
    if do_nh:
        out_specs += [pl.BlockSpec((1, 1), lambda i: (0, 0))]
        out_shape += [jax.ShapeDtypeStruct((1, 1), jnp.float32)]

    body = lambda *refs: _block_body(refs, fuse_next=fuse_next, do_nh=do_nh)
    args = [x, xi, xg, rbf] + wts + (nxt if fuse_next else [])
    return pl.pallas_call(
        body,
        grid=(grid,),
        in_specs=in_specs,
        out_specs=out_specs,
        out_shape=out_shape,
    )(*args)


def _block_weights(p):
    b = lambda v: v.reshape(1, FDIM)
    od = jnp.pad(p['odense'], ((0, 0), (0, 6)))
    k2f = jnp.pad(p['k2f'], ((0, 3), (0, 0)))
    rp = p['ires'][0]
    ap = p['ares'][0]
    op = p['ores'][0]
    return [k2f,
            rp['d1']['w'], b(rp['d1']['b']), rp['d2']['w'], b(rp['d2']['b']),
            p['dense']['w'], b(p['dense']['b']), b(p['u']),
            ap['d1']['w'], b(ap['d1']['b']), ap['d2']['w'], b(ap['d2']['b']),
            op['d1']['w'], b(op['d1']['b']), op['d2']['w'], b(op['d2']['b']),
            od, jnp.asarray(_MB), jnp.asarray(_H)]


def _run_block(x, xi, tj, rbf, idx_j, wts, nxt):
    N = x.shape[0]
    ns = N // NSLICE
    es = ns * GRP
    outs = []
    for s in range(NSLICE):
        xg = _sc_gather(tj, lax.slice(idx_j, (s * es,), ((s + 1) * es,)))
        outs.append(_block_call(x, xi, xg, rbf, wts, nxt, s * ns, ns))
    cat = lambda k: jnp.concatenate([o[k] for o in outs], axis=0)
    if nxt is not None:
        return cat(0), cat(1), cat(2), cat(3), None
    nh = sum(o[2][0, 0] for o in outs)
    return cat(0), cat(1), None, None, nh


def kernel(Z, R, idx_i, idx_j, params):
    N = Z.shape[0]
    Z = Z.astype(jnp.int32)
    idx_i = idx_i.astype(jnp.int32)
    idx_j = idx_j.astype(jnp.int32)

    d2 = _sc_edge_d2(R.astype(jnp.float32), idx_i, idx_j)
    rbf = _rbf_call(d2)

    b1, b2 = params['blocks']
    x, xi1, tj1 = _proj_call(Z, params['emb'],
                             b1['di']['w'], b1['di']['b'].reshape(1, FDIM),
                             b1['dj']['w'], b1['dj']['b'].reshape(1, FDIM))
    nxt = [b2['di']['w'], b2['di']['b'].reshape(1, FDIM),
           b2['dj']['w'], b2['dj']['b'].reshape(1, FDIM)]
    x1, out1, xi2, tj2, _ = _run_block(x, xi1, tj1, rbf, idx_j,
                                       _block_weights(b1), nxt)
    x2, out2, _, _, nh = _run_block(x1, xi2, tj2, rbf, idx_j,
                                    _block_weights(b2), None)

    e_total = out1[:, 0] + out2[:, 0]
    q_total = out1[:, 1] + out2[:, 1]
    nhloss = nh / np.float32(N * 2)
    return (e_total, q_total, nhloss)


# speedup vs baseline: 7.5156x; 1.1624x over previous
import functools

import numpy as np
import jax
import jax.numpy as jnp
from jax import lax
from jax.experimental import pallas as pl
from jax.experimental.pallas import tpu as pltpu
from jax.experimental.pallas import tpu_sc as plsc

FDIM = 128
KRBF = 5
CUTOFF = 10.0
GRP = 32
NSLICE = 5
BN = 400
BE = BN * GRP
SC_CH = 200
SC_NW = 32
LN2 = np.float32(np.log(2.0))
_CSTEP = float((np.exp(-CUTOFF) - 1.0) / (KRBF - 1))
_WIDTH = np.float32((0.5 / ((1.0 - np.exp(-CUTOFF)) / KRBF)) ** 2)


def _act(x):
    return jax.nn.softplus(x) - LN2


@functools.lru_cache(maxsize=None)
def _sc_gather_fn(V, D, B):
    per_w = B // SC_NW
    assert per_w % SC_CH == 0 and D % 128 == 0
    n_it = per_w // SC_CH
    mesh = plsc.VectorSubcoreMesh(core_axis_name="c", subcore_axis_name="s")

    @functools.partial(
        pl.kernel,
        mesh=mesh,
        out_type=jax.ShapeDtypeStruct((B, D), jnp.float32),
        scratch_types=[
            pltpu.VMEM((SC_CH,), jnp.int32),
            pltpu.VMEM((SC_CH,), jnp.int32),
            pltpu.VMEM((SC_CH, D), jnp.float32),
            pltpu.VMEM((SC_CH, D), jnp.float32),
            pltpu.SemaphoreType.DMA,
            pltpu.SemaphoreType.DMA,
        ],
    )
    def k(table_hbm, idx_hbm, out_hbm, i0, i1, r0, r1, s0, s1):
        wid = lax.axis_index("s") * 2 + lax.axis_index("c")
        base = wid * per_w
        slots = [(i0, r0, s0), (i1, r1, s1)]
        handles = [None, None]

        for c in range(n_it):
            iv, rv, sm = slots[c % 2]
            if handles[c % 2] is not None:
                handles[c % 2].wait()
                pltpu.sync_copy(
                    rv, out_hbm.at[pl.ds(base + (c - 2) * SC_CH, SC_CH)])
            pltpu.sync_copy(idx_hbm.at[pl.ds(base + c * SC_CH, SC_CH)], iv)
            handles[c % 2] = pltpu.async_copy(table_hbm.at[iv], rv, sm)
        for c in range(max(0, n_it - 2), n_it):
            iv, rv, sm = slots[c % 2]
            handles[c % 2].wait()
            pltpu.sync_copy(rv, out_hbm.at[pl.ds(base + c * SC_CH, SC_CH)])

    return k


def _sc_gather(table, idx):
    V, D = table.shape
    B = idx.shape[0]
    return _sc_gather_fn(V, D, B)(table, idx)


@functools.lru_cache(maxsize=None)
def _sc_d2_fn(N, E):
    assert E % (SC_NW * 16) == 0
    per_w = E // SC_NW
    nv = per_w // 16
    mesh = plsc.VectorSubcoreMesh(core_axis_name="c", subcore_axis_name="s")

    @functools.partial(
        pl.kernel,
        mesh=mesh,
        compiler_params=pltpu.CompilerParams(needs_layout_passes=False),
        out_type=jax.ShapeDtypeStruct((E,), jnp.float32),
        scratch_types=[
            pltpu.VMEM((N,), jnp.float32),
            pltpu.VMEM((N,), jnp.float32),
            pltpu.VMEM((N,), jnp.float32),
            pltpu.VMEM((per_w,), jnp.int32),
            pltpu.VMEM((per_w,), jnp.int32),
            pltpu.VMEM((per_w,), jnp.float32),
        ],
    )
    def k(rx_h, ry_h, rz_h, ii_h, ij_h, out_h, rx, ry, rz, iiv, ijv, d2v):
        wid = lax.axis_index("s") * 2 + lax.axis_index("c")
        base = wid * per_w
        pltpu.sync_copy(rx_h, rx)
        pltpu.sync_copy(ry_h, ry)
        pltpu.sync_copy(rz_h, rz)
        pltpu.sync_copy(ii_h.at[pl.ds(base, per_w)], iiv)
        pltpu.sync_copy(ij_h.at[pl.ds(base, per_w)], ijv)

        def body(v, carry):
            s = pl.ds(v * 16, 16)
            ii = iiv[s]
            ij = ijv[s]
            dx = plsc.load_gather(rx, [ii]) - plsc.load_gather(rx, [ij])
            dy = plsc.load_gather(ry, [ii]) - plsc.load_gather(ry, [ij])
            dz = plsc.load_gather(rz, [ii]) - plsc.load_gather(rz, [ij])
            d2v[s] = dx * dx + dy * dy + dz * dz
            return carry

        lax.fori_loop(0, nv, body, 0)
        pltpu.sync_copy(d2v, out_h.at[pl.ds(base, per_w)])

    return k


def _sc_edge_d2(R, idx_i, idx_j):
    N = R.shape[0]
    E = idx_i.shape[0]
    return _sc_d2_fn(N, E)(R[:, 0], R[:, 1], R[:, 2], idx_i, idx_j)


def _rbf_body(d2_ref, out_ref):
    d2 = d2_ref[...]
    D = jnp.sqrt(jnp.maximum(d2, 0.0))
    x = D / CUTOFF
    x3 = x ** 3
    x4 = x3 * x
    x5 = x4 * x
    cf = jnp.where(x < 1.0, 1.0 - 6.0 * x5 + 15.0 * x4 - 10.0 * x3,
                   jnp.zeros_like(x))
    eD = jnp.exp(-D)
    kk = lax.broadcasted_iota(jnp.int32, (8, 1, 1), 0)
    cen = jnp.where(kk < KRBF, 1.0 + kk.astype(jnp.float32) * _CSTEP, 0.0)
    msk = (kk < KRBF).astype(jnp.float32)
    val = cf[None] * jnp.exp(-_WIDTH * (eD[None] - cen) ** 2) * msk
    out_ref[...] = val.reshape(8, d2.size)


def _rbf_call(d2):
    E = d2.shape[0]
    return pl.pallas_call(
        _rbf_body,
        out_shape=jax.ShapeDtypeStruct((8, E), jnp.float32),
    )(d2.reshape(E // 128, 128))


def _proj_body(z_ref, emb_ref, wi_ref, bi_ref, wj_ref, bj_ref,
               x_ref, xi_ref, tj_ref):
    z = z_ref[...]
    oh = (z == lax.broadcasted_iota(jnp.int32, (z.shape[0], 32), 1))
    x = jnp.dot(oh.astype(jnp.float32), emb_ref[...],
                preferred_element_type=jnp.float32)
    xa = _act(x)
    x_ref[...] = x
    xi_ref[...] = (jnp.dot(xa, wi_ref[...], preferred_element_type=jnp.float32)
                   + bi_ref[...])
    tj_ref[...] = (jnp.dot(xa, wj_ref[...], preferred_element_type=jnp.float32)
                   + bj_ref[...])


def _proj_call(Z, emb, wi, bi, wj, bj):
    N = Z.shape[0]
    grid = N // BN
    full = lambda shape: pl.BlockSpec(shape, lambda i: tuple(0 for _ in shape))
    row = pl.BlockSpec((BN, FDIM), lambda i: (i, 0))
    embp = jnp.pad(emb, ((0, 32 - emb.shape[0]), (0, 0)))
    return pl.pallas_call(
        _proj_body,
        grid=(grid,),
        in_specs=[pl.BlockSpec((BN, 1), lambda i: (i, 0)), full((32, FDIM)),
                  full((FDIM, FDIM)), full((1, FDIM)),
                  full((FDIM, FDIM)), full((1, FDIM))],
        out_specs=[row, row, row],
        out_shape=[jax.ShapeDtypeStruct((N, FDIM), jnp.float32)] * 3,
    )(Z.reshape(N, 1), embp, wi, bi, wj, bj)


_ff = np.arange(FDIM)
_qq = np.arange(GRP)
_MB = np.ascontiguousarray(
    (_ff[:, None, None] == (4 * _qq[:, None] + _ff[None, :] // 32)[None])
    .reshape(FDIM, GRP * FDIM), np.float32)
_HS = np.ascontiguousarray((_ff % 32)[:, None] == (_ff % 32)[None, :],
                           np.float32)


def _block_body(refs, *, fuse_next, do_nh):
    (x_ref, xi_ref, xg_ref, rbf_ref, k2f_ref,
     i1w, i1b, i2w, i2b, dw, db, u_ref,
     a1w, a1b, a2w, a2b, o1w, o1b, o2w, o2b, od_ref,
     mb_ref, h_ref, one_ref) = refs[:24]
    pos = 24
    if fuse_next:
        wi2, bi2, wj2, bj2 = refs[pos:pos + 4]
        pos += 4
    xout_ref = refs[pos]
    out_ref = refs[pos + 1]
    pos += 2
    if fuse_next:
        xi2_ref, tj2_ref = refs[pos:pos + 2]
        pos += 2
    if do_nh:
        nh_ref = refs[pos]

    dot = lambda a, b: jnp.dot(a, b, preferred_element_type=jnp.float32)

    g = lax.dot_general(rbf_ref[...], k2f_ref[...],
                        (((0,), (0,)), ((), ())),
                        preferred_element_type=jnp.float32)
    xj = g * xg_ref[...]
    xi = xi_ref[...]

    B = dot(xi, mb_ref[...]).reshape(BE, FDIM)
    P = (xj * B).reshape(BN, GRP, FDIM)
    ones3 = jnp.broadcast_to(one_ref[...][None], (BN, 1, GRP))
    S = lax.dot_general(ones3, P, (((2,), (1,)), ((0,), (0,))),
                        preferred_element_type=jnp.float32)[:, 0]
    att = dot(S, h_ref[...])
    att = att - jnp.max(att, axis=1, keepdims=True)
    ea = jnp.exp(att)
    attw = ea / (jnp.sum(ea, axis=1, keepdims=True) * 0.25)
    xjagg = lax.dot_general(attw[:, None, :GRP], xj.reshape(BN, GRP, FDIM),
                            (((2,), (1,)), ((0,), (0,))),
                            preferred_element_type=jnp.float32)[:, 0]

    m = xi + xjagg
    m = m + dot(dot(_act(m), i1w[...]) + i1b[...], i2w[...]) + i2b[...]
    m = _act(m)
    xnew = u_ref[...] * x_ref[...] + dot(m, dw[...]) + db[...]
    xo = xnew + dot(dot(_act(xnew), a1w[...]) + a1b[...], a2w[...]) + a2b[...]
    h = xo + dot(dot(_act(xo), o1w[...]) + o1b[...], o2w[...]) + o2b[...]
    h = _act(h)
    out = dot(h, od_ref[...])

    xout_ref[...] = xo
    out_ref[...] = out
    if fuse_next:
        xa2 = _act(xo)
        xi2_ref[...] = dot(xa2, wi2[...]) + bi2[...]
        tj2_ref[...] = dot(xa2, wj2[...]) + bj2[...]
    if do_nh:
        o2 = out * out
        part = jnp.sum(o2 / (o2 + 1e-7)).reshape(1, 1)

        @pl.when(pl.program_id(0) == 0)
        def _():
            nh_ref[...] = jnp.zeros_like(nh_ref)

        nh_ref[...] += part


def _block_call(x, xi, xg, rbf, wts, nxt, node0, ns):
    grid = ns // BN
    ro = node0 // BN
    fuse_next = nxt is not None
    do_nh = not fuse_next

    full = lambda shape: pl.BlockSpec(shape, lambda i: tuple(0 for _ in shape))
    rowo = pl.BlockSpec((BN, FDIM), lambda i: (i + ro, 0))
    row = pl.BlockSpec((BN, FDIM), lambda i: (i, 0))
    erow = pl.BlockSpec((BE, FDIM), lambda i: (i, 0))
    rrow = pl.BlockSpec((8, BE), lambda i: (0, i + ro))
    orow = pl.BlockSpec((BN, 8), lambda i: (i, 0))

    in_specs = [rowo, rowo, erow, rrow, full((8, FDIM))]
    in_specs += [full((FDIM, FDIM)), full((1, FDIM))] * 2
    in_specs += [full((FDIM, FDIM)), full((1, FDIM)), full((1, FDIM))]
    in_specs += [full((FDIM, FDIM)), full((1, FDIM))] * 4
    in_specs += [full((FDIM, 8))]
    in_specs += [full((FDIM, GRP * FDIM)), full((FDIM, FDIM)),
                 full((1, GRP))]
    if fuse_next:
        in_specs += [full((FDIM, FDIM)), full((1, FDIM))] * 2

    out_specs = [row, orow]
    out_shape = [jax.ShapeDtypeStruct((ns, FDIM), jnp.float32),
                 jax.ShapeDtypeStruct((ns, 8), jnp.float32)]
    if fuse_next:
        out_specs += [row, row]
        out_shape += [jax.ShapeDtypeStruct((ns, FDIM), jnp.float32)] * 2
    if do_nh:
        out_specs += [pl.BlockSpec((1, 1), lambda i: (0, 0))]
        out_shape += [jax.ShapeDtypeStruct((1, 1), jnp.float32)]

    body = lambda *refs: _block_body(refs, fuse_next=fuse_next, do_nh=do_nh)
    args = [x, xi, xg, rbf] + wts + (nxt if fuse_next else [])
    return pl.pallas_call(
        body,
        grid=(grid,),
        in_specs=in_specs,
        out_specs=out_specs,
        out_shape=out_shape,
    )(*args)


def _block_weights(p):
    b = lambda v: v.reshape(1, FDIM)
    od = jnp.pad(p['odense'], ((0, 0), (0, 6)))
    k2f = jnp.pad(p['k2f'], ((0, 3), (0, 0)))
    rp = p['ires'][0]
    ap = p['ares'][0]
    op = p['ores'][0]
    return [k2f,
            rp['d1']['w'], b(rp['d1']['b']), rp['d2']['w'], b(rp['d2']['b']),
            p['dense']['w'], b(p['dense']['b']), b(p['u']),
            ap['d1']['w'], b(ap['d1']['b']), ap['d2']['w'], b(ap['d2']['b']),
            op['d1']['w'], b(op['d1']['b']), op['d2']['w'], b(op['d2']['b']),
            od, jnp.asarray(_MB), jnp.asarray(_HS),
            jnp.ones((1, GRP), jnp.float32)]


def _run_block(x, xi, tj, rbf, idx_j, wts, nxt):
    N = x.shape[0]
    ns = N // NSLICE
    es = ns * GRP
    outs = []
    for s in range(NSLICE):
        xg = _sc_gather(tj, lax.slice(idx_j, (s * es,), ((s + 1) * es,)))
        outs.append(_block_call(x, xi, xg, rbf, wts, nxt, s * ns, ns))
    cat = lambda k: jnp.concatenate([o[k] for o in outs], axis=0)
    if nxt is not None:
        return cat(0), cat(1), cat(2), cat(3), None
    nh = sum(o[2][0, 0] for o in outs)
    return cat(0), cat(1), None, None, nh


def kernel(Z, R, idx_i, idx_j, params):
    N = Z.shape[0]
    Z = Z.astype(jnp.int32)
    idx_i = idx_i.astype(jnp.int32)
    idx_j = idx_j.astype(jnp.int32)

    d2 = _sc_edge_d2(R.astype(jnp.float32), idx_i, idx_j)
    rbf = _rbf_call(d2)

    b1, b2 = params['blocks']
    x, xi1, tj1 = _proj_call(Z, params['emb'],
                             b1['di']['w'], b1['di']['b'].reshape(1, FDIM),
                             b1['dj']['w'], b1['dj']['b'].reshape(1, FDIM))
    nxt = [b2['di']['w'], b2['di']['b'].reshape(1, FDIM),
           b2['dj']['w'], b2['dj']['b'].reshape(1, FDIM)]
    x1, out1, xi2, tj2, _ = _run_block(x, xi1, tj1, rbf, idx_j,
                                       _block_weights(b1), nxt)
    x2, out2, _, _, nh = _run_block(x1, xi2, tj2, rbf, idx_j,
                                    _block_weights(b2), None)

    e_total = out1[:, 0] + out2[:, 0]
    q_total = out1[:, 1] + out2[:, 1]
    nhloss = nh / np.float32(N * 2)
    return (e_total, q_total, nhloss)
